# Initial kernel scaffold; baseline (speedup 1.0000x reference)
import jax
import jax.numpy as jnp
from jax.experimental import pallas as pl


def kernel(frac, params, src):
    raise NotImplementedError("write your pallas kernel here")


# speedup vs baseline: 1.2857x; 1.2857x over previous
import functools

import jax
import jax.numpy as jnp
import numpy as np
from jax.experimental import pallas as pl

D_MODEL = 1024
N_HEADS = 16
HEAD_DIM = 64
N_EXPERTS = 8
D_FF = 2048
N_BASIS = 10
DEGREE = 3
VOCAB = 120
FEAT = 200
B = 512
T = 8
N_TOK = B * T

_base = np.linspace(0.0, 1.0, N_BASIS + DEGREE + 1 - 2 * DEGREE)
_KNOTS = np.concatenate(
    [np.repeat(_base[:1], DEGREE), _base, np.repeat(_base[-1:], DEGREE)]
).astype(np.float64)


def _bspline_basis(f):
    nk = _KNOTS.shape[0]
    Bp = [
        jnp.where((f >= float(_KNOTS[i])) & (f < float(_KNOTS[i + 1])), 1.0, 0.0)
        for i in range(nk - 1)
    ]
    for d in range(1, DEGREE + 1):
        Bc = []
        for i in range(nk - d - 1):
            den1 = float(_KNOTS[i + d] - _KNOTS[i])
            den2 = float(_KNOTS[i + d + 1] - _KNOTS[i + 1])
            t = jnp.zeros_like(f)
            if den1 != 0.0:
                t = t + (f - float(_KNOTS[i])) / den1 * Bp[i]
            if den2 != 0.0:
                t = t + (float(_KNOTS[i + d + 1]) - f) / den2 * Bp[i + 1]
            Bc.append(t)
        Bp = Bc
    return jnp.concatenate(Bp, axis=1)


def _embed_body(src_ref, frac_ref, cbfv_ref, we_ref, be_ref, wpe_ref, bpe_ref,
                wple_ref, bple_ref, sc_ref, out_ref):
    src = src_ref[...]
    rows = src.shape[0]
    oh = (src == jax.lax.broadcasted_iota(jnp.int32, (rows, VOCAB), 1)).astype(
        jnp.float32)
    feats = jnp.dot(oh, cbfv_ref[...], preferred_element_type=jnp.float32)
    x = jnp.dot(feats, we_ref[...], preferred_element_type=jnp.float32)
    x = x + be_ref[...]
    emb_scaler = sc_ref[0, 0]
    pos_scaler = sc_ref[0, 1]
    pos_scaler_log = sc_ref[0, 2]
    x = x * jnp.exp2(emb_scaler)
    pe_scaler = jnp.exp2((1.0 - pos_scaler) ** 2)
    ple_scaler = jnp.exp2((1.0 - pos_scaler_log) ** 2)

    f = jnp.clip(frac_ref[...], 1e-9, 1.0)
    basis = _bspline_basis(f)
    pe = (jnp.dot(basis, wpe_ref[...], preferred_element_type=jnp.float32)
          + bpe_ref[...]) * pe_scaler
    f2 = jnp.clip(0.0025 * jnp.log2(f) ** 2, 0.0, 1.0)
    basis2 = _bspline_basis(f2)
    ple = (jnp.dot(basis2, wple_ref[...], preferred_element_type=jnp.float32)
           + bple_ref[...]) * ple_scaler
    out_ref[...] = x + jnp.concatenate([pe, ple], axis=1)


def _embed(src2d, frac2d, p):
    blk = 512
    grid = (N_TOK // blk,)
    scalars = jnp.stack([p['emb_scaler'], p['pos_scaler'],
                         p['pos_scaler_log']]).reshape(1, 3)
    half = D_MODEL // 2
    return pl.pallas_call(
        _embed_body,
        grid=grid,
        in_specs=[
            pl.BlockSpec((blk, 1), lambda i: (i, 0)),
            pl.BlockSpec((blk, 1), lambda i: (i, 0)),
            pl.BlockSpec((VOCAB, FEAT), lambda i: (0, 0)),
            pl.BlockSpec((FEAT, D_MODEL), lambda i: (0, 0)),
            pl.BlockSpec((1, D_MODEL), lambda i: (0, 0)),
            pl.BlockSpec((N_BASIS, half), lambda i: (0, 0)),
            pl.BlockSpec((1, half), lambda i: (0, 0)),
            pl.BlockSpec((N_BASIS, half), lambda i: (0, 0)),
            pl.BlockSpec((1, half), lambda i: (0, 0)),
            pl.BlockSpec((1, 3), lambda i: (0, 0)),
        ],
        out_specs=pl.BlockSpec((blk, D_MODEL), lambda i: (i, 0)),
        out_shape=jax.ShapeDtypeStruct((N_TOK, D_MODEL), jnp.float32),
    )(src2d, frac2d, p['cbfv'], p['We'], p['be'].reshape(1, -1),
      p['W_pe'], p['b_pe'].reshape(1, -1), p['W_ple'], p['b_ple'].reshape(1, -1),
      scalars)


ATT_ROWS = 256


def _attn_body(x_ref, frac_ref, fracr_ref, wqkv_ref, bqkv_ref, alpha_ref,
               out_ref):
    x = x_ref[...]
    qkv = jnp.dot(x, wqkv_ref[...], preferred_element_type=jnp.float32)
    qkv = qkv + bqkv_ref[...]
    fcol = frac_ref[...]
    frow = fracr_ref[0]
    alpha = alpha_ref[0, 0]
    R = ATT_ROWS
    diff = fcol - frow
    bias = alpha * (jnp.log1p(jnp.abs(diff)) * jnp.sign(diff))
    ii = jax.lax.broadcasted_iota(jnp.int32, (R, R), 0)
    jj = jax.lax.broadcasted_iota(jnp.int32, (R, R), 1)
    same = (ii // T) == (jj // T)
    keyok = frow != 0.0
    valid = same & keyok
    scale = HEAD_DIM ** -0.5
    for h in range(N_HEADS):
        q = qkv[:, h * HEAD_DIM:(h + 1) * HEAD_DIM]
        k = qkv[:, D_MODEL + h * HEAD_DIM:D_MODEL + (h + 1) * HEAD_DIM]
        v = qkv[:, 2 * D_MODEL + h * HEAD_DIM:2 * D_MODEL + (h + 1) * HEAD_DIM]
        s = jax.lax.dot_general(q, k, (((1,), (1,)), ((), ())),
                                preferred_element_type=jnp.float32) * scale
        s = jnp.where(valid, s + bias, -1e30)
        m = jnp.max(s, axis=1, keepdims=True)
        e = jnp.exp(s - m)
        pr = e / jnp.sum(e, axis=1, keepdims=True)
        ctx = jnp.dot(pr, v, preferred_element_type=jnp.float32)
        out_ref[:, h * HEAD_DIM:(h + 1) * HEAD_DIM] = ctx


def _attn(x, frac2d, fracr, p, wqkv, bqkv):
    grid = (N_TOK // ATT_ROWS,)
    return pl.pallas_call(
        _attn_body,
        grid=grid,
        in_specs=[
            pl.BlockSpec((ATT_ROWS, D_MODEL), lambda i: (i, 0)),
            pl.BlockSpec((ATT_ROWS, 1), lambda i: (i, 0)),
            pl.BlockSpec((1, 1, ATT_ROWS), lambda i: (i, 0, 0)),
            pl.BlockSpec((D_MODEL, 3 * D_MODEL), lambda i: (0, 0)),
            pl.BlockSpec((1, 3 * D_MODEL), lambda i: (0, 0)),
            pl.BlockSpec((1, 1), lambda i: (0, 0)),
        ],
        out_specs=pl.BlockSpec((ATT_ROWS, D_MODEL), lambda i: (i, 0)),
        out_shape=jax.ShapeDtypeStruct((N_TOK, D_MODEL), jnp.float32),
    )(x, frac2d, fracr, wqkv, bqkv, p['alpha'].reshape(1, 1))


def _post_body(ctx_ref, xin_ref, wo_ref, bo_ref, g1_ref, b1_ref, wg_ref,
               bg_ref, x1_ref, comb_ref):
    y = jnp.dot(ctx_ref[...], wo_ref[...], preferred_element_type=jnp.float32)
    y = y + bo_ref[...] + xin_ref[...]
    m = jnp.mean(y, axis=1, keepdims=True)
    v = jnp.mean((y - m) ** 2, axis=1, keepdims=True)
    x1 = (y - m) / jnp.sqrt(v + 1e-5) * g1_ref[...] + b1_ref[...]
    x1_ref[...] = x1
    logits = jnp.dot(x1, wg_ref[...], preferred_element_type=jnp.float32)
    logits = logits + bg_ref[...]
    lm = jnp.max(logits, axis=1, keepdims=True)
    le = jnp.exp(logits - lm)
    probs = le / jnp.sum(le, axis=1, keepdims=True)
    rows = probs.shape[0]
    lane = jax.lax.broadcasted_iota(jnp.int32, (rows, N_EXPERTS), 1)
    w1 = jnp.max(probs, axis=1, keepdims=True)
    i1 = jnp.min(jnp.where(probs == w1, lane, N_EXPERTS), axis=1, keepdims=True)
    probs2 = jnp.where(lane == i1, -1.0, probs)
    w2 = jnp.max(probs2, axis=1, keepdims=True)
    i2 = jnp.min(jnp.where(probs2 == w2, lane, N_EXPERTS), axis=1, keepdims=True)
    comb = (jnp.where(lane == i1, w1, 0.0) + jnp.where(lane == i2, w2, 0.0))
    comb_ref[...] = comb


def _post(ctx, xin, p, wo, bo):
    blk = 512
    grid = (N_TOK // blk,)
    return pl.pallas_call(
        _post_body,
        grid=grid,
        in_specs=[
            pl.BlockSpec((blk, D_MODEL), lambda i: (i, 0)),
            pl.BlockSpec((blk, D_MODEL), lambda i: (i, 0)),
            pl.BlockSpec((D_MODEL, D_MODEL), lambda i: (0, 0)),
            pl.BlockSpec((1, D_MODEL), lambda i: (0, 0)),
            pl.BlockSpec((1, D_MODEL), lambda i: (0, 0)),
            pl.BlockSpec((1, D_MODEL), lambda i: (0, 0)),
            pl.BlockSpec((D_MODEL, N_EXPERTS), lambda i: (0, 0)),
            pl.BlockSpec((1, N_EXPERTS), lambda i: (0, 0)),
        ],
        out_specs=[
            pl.BlockSpec((blk, D_MODEL), lambda i: (i, 0)),
            pl.BlockSpec((blk, N_EXPERTS), lambda i: (i, 0)),
        ],
        out_shape=[
            jax.ShapeDtypeStruct((N_TOK, D_MODEL), jnp.float32),
            jax.ShapeDtypeStruct((N_TOK, N_EXPERTS), jnp.float32),
        ],
    )(ctx, xin, wo, bo, p['g1'].reshape(1, -1), p['b1'].reshape(1, -1),
      p['Wg'], p['bg'].reshape(1, -1))


def _moe_body(x_ref, comb_ref, we1_ref, be1_ref, we2_ref, be2_ref, out_ref):
    e = pl.program_id(1)
    x = x_ref[...]
    h = jnp.dot(x, we1_ref[0], preferred_element_type=jnp.float32)
    h = jnp.maximum(h + be1_ref[0], 0.0)
    y = jnp.dot(h, we2_ref[0], preferred_element_type=jnp.float32)
    y = y + be2_ref[0]
    comb = comb_ref[...]
    lane = jax.lax.broadcasted_iota(jnp.int32, comb.shape, 1)
    cw = jnp.sum(jnp.where(lane == e, comb, 0.0), axis=1, keepdims=True)

    @pl.when(e == 0)
    def _init():
        out_ref[...] = cw * y

    @pl.when(e > 0)
    def _acc():
        out_ref[...] = out_ref[...] + cw * y


def _moe_dense(x1, comb, p):
    blk = 512
    nb = N_TOK // blk
    return pl.pallas_call(
        _moe_body,
        grid=(nb, N_EXPERTS),
        in_specs=[
            pl.BlockSpec((blk, D_MODEL), lambda b, e: (b, 0)),
            pl.BlockSpec((blk, N_EXPERTS), lambda b, e: (b, 0)),
            pl.BlockSpec((1, D_MODEL, D_FF), lambda b, e: (e, 0, 0)),
            pl.BlockSpec((1, 1, D_FF), lambda b, e: (e, 0, 0)),
            pl.BlockSpec((1, D_FF, D_MODEL), lambda b, e: (e, 0, 0)),
            pl.BlockSpec((1, 1, D_MODEL), lambda b, e: (e, 0, 0)),
        ],
        out_specs=pl.BlockSpec((blk, D_MODEL), lambda b, e: (b, 0)),
        out_shape=jax.ShapeDtypeStruct((N_TOK, D_MODEL), jnp.float32),
    )(x1, comb, p['We1'], p['be1'].reshape(N_EXPERTS, 1, D_FF),
      p['We2'], p['be2'].reshape(N_EXPERTS, 1, D_MODEL))


def _ln2_body(f_ref, xres_ref, g_ref, b_ref, mask_ref, out_ref):
    y = f_ref[...] + xres_ref[...]
    m = jnp.mean(y, axis=1, keepdims=True)
    v = jnp.mean((y - m) ** 2, axis=1, keepdims=True)
    x2 = (y - m) / jnp.sqrt(v + 1e-5) * g_ref[...] + b_ref[...]
    out_ref[...] = x2 * mask_ref[...]


def _ln2(f, xres, p, mask):
    blk = 512
    return pl.pallas_call(
        _ln2_body,
        grid=(N_TOK // blk,),
        in_specs=[
            pl.BlockSpec((blk, D_MODEL), lambda i: (i, 0)),
            pl.BlockSpec((blk, D_MODEL), lambda i: (i, 0)),
            pl.BlockSpec((1, D_MODEL), lambda i: (0, 0)),
            pl.BlockSpec((1, D_MODEL), lambda i: (0, 0)),
            pl.BlockSpec((blk, 1), lambda i: (i, 0)),
        ],
        out_specs=pl.BlockSpec((blk, D_MODEL), lambda i: (i, 0)),
        out_shape=jax.ShapeDtypeStruct((N_TOK, D_MODEL), jnp.float32),
    )(f, xres, p['g2'].reshape(1, -1), p['b2'].reshape(1, -1), mask)


def kernel(frac, params, src):
    p = params
    frac2d = frac.reshape(N_TOK, 1)
    fracr = frac.reshape(N_TOK // ATT_ROWS, 1, ATT_ROWS)
    src2d = src.reshape(N_TOK, 1).astype(jnp.int32)
    wqkv = jnp.concatenate([p['Wq'], p['Wk'], p['Wv']], axis=1)
    bqkv = jnp.concatenate([p['bq'], p['bk'], p['bv']]).reshape(1, -1)
    ones = jnp.ones((N_TOK, 1), jnp.float32)
    finalmask = (frac2d != 0.0).astype(jnp.float32)

    x = _embed(src2d, frac2d, p)
    for layer in range(3):
        ctx = _attn(x, frac2d, fracr, p, wqkv, bqkv)
        x1, comb = _post(ctx, x, p, p['Wo'], p['bo'].reshape(1, -1))
        f = _moe_dense(x1, comb, p)
        x = _ln2(f, x1, p, finalmask if layer == 2 else ones)
    return x.reshape(B, T, D_MODEL)


# speedup vs baseline: 1.7538x; 1.3641x over previous
import functools

import jax
import jax.numpy as jnp
import numpy as np
from jax.experimental import pallas as pl
from jax.experimental.pallas import tpu as pltpu
from jax.experimental.pallas import tpu_sc as plsc

D_MODEL = 1024
N_HEADS = 16
HEAD_DIM = 64
N_EXPERTS = 8
D_FF = 2048
N_BASIS = 10
DEGREE = 3
VOCAB = 120
FEAT = 200
B = 512
T = 8
N_TOK = B * T
N_SLOT = 2 * N_TOK
GBLK = 256
NGB = N_SLOT // GBLK
NSTEP = NGB + N_EXPERTS

_NC, _NS = 2, 16
_NW = _NC * _NS

_base = np.linspace(0.0, 1.0, N_BASIS + DEGREE + 1 - 2 * DEGREE)
_KNOTS = np.concatenate(
    [np.repeat(_base[:1], DEGREE), _base, np.repeat(_base[-1:], DEGREE)]
).astype(np.float64)


def _bspline_basis(f):
    nk = _KNOTS.shape[0]
    Bp = [
        jnp.where((f >= float(_KNOTS[i])) & (f < float(_KNOTS[i + 1])), 1.0, 0.0)
        for i in range(nk - 1)
    ]
    for d in range(1, DEGREE + 1):
        Bc = []
        for i in range(nk - d - 1):
            den1 = float(_KNOTS[i + d] - _KNOTS[i])
            den2 = float(_KNOTS[i + d + 1] - _KNOTS[i + 1])
            t = jnp.zeros_like(f)
            if den1 != 0.0:
                t = t + (f - float(_KNOTS[i])) / den1 * Bp[i]
            if den2 != 0.0:
                t = t + (float(_KNOTS[i + d + 1]) - f) / den2 * Bp[i + 1]
            Bc.append(t)
        Bp = Bc
    return jnp.concatenate(Bp, axis=1)


def _embed_body(src_ref, frac_ref, cbfv_ref, we_ref, be_ref, wpe_ref, bpe_ref,
                wple_ref, bple_ref, sc_ref, out_ref):
    src = src_ref[...]
    rows = src.shape[0]
    oh = (src == jax.lax.broadcasted_iota(jnp.int32, (rows, VOCAB), 1)).astype(
        jnp.float32)
    feats = jnp.dot(oh, cbfv_ref[...], preferred_element_type=jnp.float32)
    x = jnp.dot(feats, we_ref[...], preferred_element_type=jnp.float32)
    x = x + be_ref[...]
    emb_scaler = sc_ref[0, 0]
    pos_scaler = sc_ref[0, 1]
    pos_scaler_log = sc_ref[0, 2]
    x = x * jnp.exp2(emb_scaler)
    pe_scaler = jnp.exp2((1.0 - pos_scaler) ** 2)
    ple_scaler = jnp.exp2((1.0 - pos_scaler_log) ** 2)

    f = jnp.clip(frac_ref[...], 1e-9, 1.0)
    basis = _bspline_basis(f)
    pe = (jnp.dot(basis, wpe_ref[...], preferred_element_type=jnp.float32)
          + bpe_ref[...]) * pe_scaler
    f2 = jnp.clip(0.0025 * jnp.log2(f) ** 2, 0.0, 1.0)
    basis2 = _bspline_basis(f2)
    ple = (jnp.dot(basis2, wple_ref[...], preferred_element_type=jnp.float32)
           + bple_ref[...]) * ple_scaler
    out_ref[...] = x + jnp.concatenate([pe, ple], axis=1)


def _embed(src2d, frac2d, p):
    blk = 512
    grid = (N_TOK // blk,)
    scalars = jnp.stack([p['emb_scaler'], p['pos_scaler'],
                         p['pos_scaler_log']]).reshape(1, 3)
    half = D_MODEL // 2
    return pl.pallas_call(
        _embed_body,
        grid=grid,
        in_specs=[
            pl.BlockSpec((blk, 1), lambda i: (i, 0)),
            pl.BlockSpec((blk, 1), lambda i: (i, 0)),
            pl.BlockSpec((VOCAB, FEAT), lambda i: (0, 0)),
            pl.BlockSpec((FEAT, D_MODEL), lambda i: (0, 0)),
            pl.BlockSpec((1, D_MODEL), lambda i: (0, 0)),
            pl.BlockSpec((N_BASIS, half), lambda i: (0, 0)),
            pl.BlockSpec((1, half), lambda i: (0, 0)),
            pl.BlockSpec((N_BASIS, half), lambda i: (0, 0)),
            pl.BlockSpec((1, half), lambda i: (0, 0)),
            pl.BlockSpec((1, 3), lambda i: (0, 0)),
        ],
        out_specs=pl.BlockSpec((blk, D_MODEL), lambda i: (i, 0)),
        out_shape=jax.ShapeDtypeStruct((N_TOK, D_MODEL), jnp.float32),
    )(src2d, frac2d, p['cbfv'], p['We'], p['be'].reshape(1, -1),
      p['W_pe'], p['b_pe'].reshape(1, -1), p['W_ple'], p['b_ple'].reshape(1, -1),
      scalars)


ATT_ROWS = 256


def _attn_body(x_ref, frac_ref, fracr_ref, wqkv_ref, bqkv_ref, alpha_ref,
               out_ref):
    x = x_ref[...]
    qkv = jnp.dot(x, wqkv_ref[...], preferred_element_type=jnp.float32)
    qkv = qkv + bqkv_ref[...]
    fcol = frac_ref[...]
    frow = fracr_ref[0]
    alpha = alpha_ref[0, 0]
    R = ATT_ROWS
    diff = fcol - frow
    bias = alpha * (jnp.log1p(jnp.abs(diff)) * jnp.sign(diff))
    ii = jax.lax.broadcasted_iota(jnp.int32, (R, R), 0)
    jj = jax.lax.broadcasted_iota(jnp.int32, (R, R), 1)
    same = (ii // T) == (jj // T)
    keyok = frow != 0.0
    valid = same & keyok
    scale = HEAD_DIM ** -0.5
    for h in range(N_HEADS):
        q = qkv[:, h * HEAD_DIM:(h + 1) * HEAD_DIM]
        k = qkv[:, D_MODEL + h * HEAD_DIM:D_MODEL + (h + 1) * HEAD_DIM]
        v = qkv[:, 2 * D_MODEL + h * HEAD_DIM:2 * D_MODEL + (h + 1) * HEAD_DIM]
        s = jax.lax.dot_general(q, k, (((1,), (1,)), ((), ())),
                                preferred_element_type=jnp.float32) * scale
        s = jnp.where(valid, s + bias, -1e30)
        m = jnp.max(s, axis=1, keepdims=True)
        e = jnp.exp(s - m)
        pr = e / jnp.sum(e, axis=1, keepdims=True)
        ctx = jnp.dot(pr, v, preferred_element_type=jnp.float32)
        out_ref[:, h * HEAD_DIM:(h + 1) * HEAD_DIM] = ctx


def _attn(x, frac2d, fracr, p, wqkv, bqkv):
    grid = (N_TOK // ATT_ROWS,)
    return pl.pallas_call(
        _attn_body,
        grid=grid,
        in_specs=[
            pl.BlockSpec((ATT_ROWS, D_MODEL), lambda i: (i, 0)),
            pl.BlockSpec((ATT_ROWS, 1), lambda i: (i, 0)),
            pl.BlockSpec((1, 1, ATT_ROWS), lambda i: (i, 0, 0)),
            pl.BlockSpec((D_MODEL, 3 * D_MODEL), lambda i: (0, 0)),
            pl.BlockSpec((1, 3 * D_MODEL), lambda i: (0, 0)),
            pl.BlockSpec((1, 1), lambda i: (0, 0)),
        ],
        out_specs=pl.BlockSpec((ATT_ROWS, D_MODEL), lambda i: (i, 0)),
        out_shape=jax.ShapeDtypeStruct((N_TOK, D_MODEL), jnp.float32),
    )(x, frac2d, fracr, wqkv, bqkv, p['alpha'].reshape(1, 1))


def _post_body(ctx_ref, xin_ref, wo_ref, bo_ref, g1_ref, b1_ref, wg_ref,
               bg_ref, x1_ref, i1_ref, i2_ref, w1_ref, w2_ref):
    y = jnp.dot(ctx_ref[...], wo_ref[...], preferred_element_type=jnp.float32)
    y = y + bo_ref[...] + xin_ref[...]
    m = jnp.mean(y, axis=1, keepdims=True)
    v = jnp.mean((y - m) ** 2, axis=1, keepdims=True)
    x1 = (y - m) / jnp.sqrt(v + 1e-5) * g1_ref[...] + b1_ref[...]
    x1_ref[...] = x1
    logits = jnp.dot(x1, wg_ref[...], preferred_element_type=jnp.float32)
    logits = logits + bg_ref[...]
    lm = jnp.max(logits, axis=1, keepdims=True)
    le = jnp.exp(logits - lm)
    probs = le / jnp.sum(le, axis=1, keepdims=True)
    rows = probs.shape[0]
    lane = jax.lax.broadcasted_iota(jnp.int32, (rows, N_EXPERTS), 1)
    w1 = jnp.max(probs, axis=1, keepdims=True)
    i1 = jnp.min(jnp.where(probs == w1, lane, N_EXPERTS), axis=1, keepdims=True)
    probs2 = jnp.where(lane == i1, -1.0, probs)
    w2 = jnp.max(probs2, axis=1, keepdims=True)
    i2 = jnp.min(jnp.where(probs2 == w2, lane, N_EXPERTS), axis=1, keepdims=True)
    i1_ref[...] = i1
    i2_ref[...] = i2
    w1_ref[...] = w1
    w2_ref[...] = w2


def _post(ctx, xin, p, wo, bo):
    blk = 512
    grid = (N_TOK // blk,)
    return pl.pallas_call(
        _post_body,
        grid=grid,
        in_specs=[
            pl.BlockSpec((blk, D_MODEL), lambda i: (i, 0)),
            pl.BlockSpec((blk, D_MODEL), lambda i: (i, 0)),
            pl.BlockSpec((D_MODEL, D_MODEL), lambda i: (0, 0)),
            pl.BlockSpec((1, D_MODEL), lambda i: (0, 0)),
            pl.BlockSpec((1, D_MODEL), lambda i: (0, 0)),
            pl.BlockSpec((1, D_MODEL), lambda i: (0, 0)),
            pl.BlockSpec((D_MODEL, N_EXPERTS), lambda i: (0, 0)),
            pl.BlockSpec((1, N_EXPERTS), lambda i: (0, 0)),
        ],
        out_specs=[
            pl.BlockSpec((blk, D_MODEL), lambda i: (i, 0)),
            pl.BlockSpec((blk, 1), lambda i: (i, 0)),
            pl.BlockSpec((blk, 1), lambda i: (i, 0)),
            pl.BlockSpec((blk, 1), lambda i: (i, 0)),
            pl.BlockSpec((blk, 1), lambda i: (i, 0)),
        ],
        out_shape=[
            jax.ShapeDtypeStruct((N_TOK, D_MODEL), jnp.float32),
            jax.ShapeDtypeStruct((N_TOK, 1), jnp.int32),
            jax.ShapeDtypeStruct((N_TOK, 1), jnp.int32),
            jax.ShapeDtypeStruct((N_TOK, 1), jnp.float32),
            jax.ShapeDtypeStruct((N_TOK, 1), jnp.float32),
        ],
    )(ctx, xin, wo, bo, p['g1'].reshape(1, -1), p['b1'].reshape(1, -1),
      p['Wg'], p['bg'].reshape(1, -1))


def _cumsum_rows(a):
    n, w = a.shape
    sh = 1
    while sh < n:
        a = a + jnp.concatenate(
            [jnp.zeros((sh, w), a.dtype), a[:-sh]], axis=0)
        sh *= 2
    return a


def _pos_body(i1_ref, i2_ref, p1_ref, p2_ref, meta_ref):
    lane = jax.lax.broadcasted_iota(jnp.int32, (N_TOK, N_EXPERTS), 1)
    h1 = (i1_ref[...] == lane).astype(jnp.int32)
    h2 = (i2_ref[...] == lane).astype(jnp.int32)
    c1 = _cumsum_rows(h1)
    c2 = _cumsum_rows(h2)
    cnt1 = c1[N_TOK - 1:N_TOK, :]
    cnt2 = c2[N_TOK - 1:N_TOK, :]
    counts = cnt1 + cnt2
    lane8 = jax.lax.broadcasted_iota(jnp.int32, (1, N_EXPERTS), 1)

    offs_row = jnp.zeros((1, N_EXPERTS), jnp.int32)
    off = jnp.int32(0)
    off_e = []
    cnt_e = []
    cnt1_e = []
    for e in range(N_EXPERTS):
        ce = jnp.sum(jnp.where(lane8 == e, counts, 0))
        c1e = jnp.sum(jnp.where(lane8 == e, cnt1, 0))
        off_e.append(off)
        cnt_e.append(ce)
        cnt1_e.append(c1e)
        offs_row = offs_row + jnp.where(lane8 == e, off, 0)
        off = off + ce

    cnt1_row = cnt1
    p1_ref[...] = jnp.sum(h1 * (offs_row + c1 - h1), axis=1, keepdims=True)
    p2_ref[...] = jnp.sum(h2 * (offs_row + cnt1_row + c2 - h2), axis=1,
                          keepdims=True)

    lane_s = jax.lax.broadcasted_iota(jnp.int32, (1, NSTEP), 1)
    rb_row = jnp.zeros((1, NSTEP), jnp.int32)
    e_row = jnp.zeros((1, NSTEP), jnp.int32)
    st_row = jnp.zeros((1, NSTEP), jnp.int32)
    en_row = jnp.zeros((1, NSTEP), jnp.int32)
    any_row = jnp.zeros((1, NSTEP), jnp.int32)
    cum = jnp.int32(0)
    for e in range(N_EXPERTS):
        start = off_e[e]
        end = off_e[e] + cnt_e[e]
        nonempty = cnt_e[e] > 0
        fb = start // GBLK
        lb = jnp.where(nonempty, (end - 1) // GBLK, 0)
        nb = jnp.where(nonempty, lb - fb + 1, 0)
        active = (lane_s >= cum) & (lane_s < cum + nb)
        rb_here = fb + (lane_s - cum)
        rb_row = rb_row + jnp.where(active, rb_here, 0)
        e_row = e_row + jnp.where(active, e, 0)
        st_row = st_row + jnp.where(active, jnp.maximum(start, rb_here * GBLK), 0)
        en_row = en_row + jnp.where(active, jnp.minimum(end, (rb_here + 1) * GBLK), 0)
        any_row = any_row + active.astype(jnp.int32)
        cum = cum + nb
    rb_row = jnp.where(any_row > 0, rb_row, NGB - 1)
    prev = jnp.concatenate(
        [jnp.full((1, 1), -1, jnp.int32), rb_row[:, :NSTEP - 1]], axis=1)
    first_row = (rb_row != prev).astype(jnp.int32)
    meta_ref[...] = jnp.concatenate(
        [rb_row, e_row, first_row, st_row, en_row], axis=0)


def _positions(i1, i2):
    return pl.pallas_call(
        _pos_body,
        grid=(1,),
        in_specs=[
            pl.BlockSpec((N_TOK, 1), lambda i: (0, 0)),
            pl.BlockSpec((N_TOK, 1), lambda i: (0, 0)),
        ],
        out_specs=[
            pl.BlockSpec((N_TOK, 1), lambda i: (0, 0)),
            pl.BlockSpec((N_TOK, 1), lambda i: (0, 0)),
            pl.BlockSpec((5, NSTEP), lambda i: (0, 0)),
        ],
        out_shape=[
            jax.ShapeDtypeStruct((N_TOK, 1), jnp.int32),
            jax.ShapeDtypeStruct((N_TOK, 1), jnp.int32),
            jax.ShapeDtypeStruct((5, NSTEP), jnp.int32),
        ],
    )(i1, i2)


_SC_CH = 64


def _sc_mesh():
    return plsc.VectorSubcoreMesh(core_axis_name="c", subcore_axis_name="s",
                                  num_cores=_NC, num_subcores=_NS)


def _sc_dispatch(x1, p1f, p2f):
    tok_per_w = N_TOK // _NW

    def body(x_hbm, p1_hbm, p2_hbm, xs_hbm, idx_v, rows_v, sem):
        wid = jax.lax.axis_index("s") * _NC + jax.lax.axis_index("c")
        base0 = wid * tok_per_w
        for c in range(tok_per_w // _SC_CH):
            base = base0 + c * _SC_CH
            pltpu.sync_copy(x_hbm.at[pl.ds(base, _SC_CH)], rows_v)
            pltpu.sync_copy(p1_hbm.at[pl.ds(base, _SC_CH)], idx_v)
            pltpu.async_copy(rows_v, xs_hbm.at[idx_v], sem).wait()
            pltpu.sync_copy(p2_hbm.at[pl.ds(base, _SC_CH)], idx_v)
            pltpu.async_copy(rows_v, xs_hbm.at[idx_v], sem).wait()

    f = pl.kernel(
        body,
        out_type=jax.ShapeDtypeStruct((N_SLOT, D_MODEL), jnp.float32),
        mesh=_sc_mesh(),
        scratch_types=[
            pltpu.VMEM((_SC_CH,), jnp.int32),
            pltpu.VMEM((_SC_CH, D_MODEL), jnp.float32),
            pltpu.SemaphoreType.DMA,
        ],
    )
    return f(x1, p1f, p2f)


def _sc_gather2(ys, p1f, p2f):
    tok_per_w = N_TOK // _NW

    def body(ys_hbm, p1_hbm, p2_hbm, o1_hbm, o2_hbm, idx_v, rows_v, sem):
        wid = jax.lax.axis_index("s") * _NC + jax.lax.axis_index("c")
        base0 = wid * tok_per_w
        for c in range(tok_per_w // _SC_CH):
            base = base0 + c * _SC_CH
            pltpu.sync_copy(p1_hbm.at[pl.ds(base, _SC_CH)], idx_v)
            pltpu.async_copy(ys_hbm.at[idx_v], rows_v, sem).wait()
            pltpu.sync_copy(rows_v, o1_hbm.at[pl.ds(base, _SC_CH)])
            pltpu.sync_copy(p2_hbm.at[pl.ds(base, _SC_CH)], idx_v)
            pltpu.async_copy(ys_hbm.at[idx_v], rows_v, sem).wait()
            pltpu.sync_copy(rows_v, o2_hbm.at[pl.ds(base, _SC_CH)])

    f = pl.kernel(
        body,
        out_type=[
            jax.ShapeDtypeStruct((N_TOK, D_MODEL), jnp.float32),
            jax.ShapeDtypeStruct((N_TOK, D_MODEL), jnp.float32),
        ],
        mesh=_sc_mesh(),
        scratch_types=[
            pltpu.VMEM((_SC_CH,), jnp.int32),
            pltpu.VMEM((_SC_CH, D_MODEL), jnp.float32),
            pltpu.SemaphoreType.DMA,
        ],
    )
    return f(ys, p1f, p2f)


def _gmm_body(meta_ref, xs_ref, we1_ref, be1_ref, we2_ref, be2_ref, ys_ref):
    s = pl.program_id(0)
    rb = meta_ref[0, s]
    first = meta_ref[2, s]
    start = meta_ref[3, s]
    end = meta_ref[4, s]
    x = xs_ref[...]
    h = jnp.dot(x, we1_ref[0], preferred_element_type=jnp.float32)
    h = jnp.maximum(h + be1_ref[0], 0.0)
    y = jnp.dot(h, we2_ref[0], preferred_element_type=jnp.float32)
    y = y + be2_ref[0]
    gi = rb * GBLK + jax.lax.broadcasted_iota(jnp.int32, (GBLK, 1), 0)
    rowmask = (gi >= start) & (gi < end)
    contrib = jnp.where(rowmask, y, 0.0)

    @pl.when(first == 1)
    def _init():
        ys_ref[...] = contrib

    @pl.when(first == 0)
    def _acc():
        ys_ref[...] = ys_ref[...] + contrib


def _gmm(meta, xs, p):
    grid_spec = pltpu.PrefetchScalarGridSpec(
        num_scalar_prefetch=1,
        grid=(NSTEP,),
        in_specs=[
            pl.BlockSpec((GBLK, D_MODEL), lambda s, m: (m[0, s], 0)),
            pl.BlockSpec((1, D_MODEL, D_FF), lambda s, m: (m[1, s], 0, 0)),
            pl.BlockSpec((1, 1, D_FF), lambda s, m: (m[1, s], 0, 0)),
            pl.BlockSpec((1, D_FF, D_MODEL), lambda s, m: (m[1, s], 0, 0)),
            pl.BlockSpec((1, 1, D_MODEL), lambda s, m: (m[1, s], 0, 0)),
        ],
        out_specs=pl.BlockSpec((GBLK, D_MODEL), lambda s, m: (m[0, s], 0)),
    )
    return pl.pallas_call(
        _gmm_body,
        grid_spec=grid_spec,
        out_shape=jax.ShapeDtypeStruct((N_SLOT, D_MODEL), jnp.float32),
    )(meta, xs, p['We1'], p['be1'].reshape(N_EXPERTS, 1, D_FF),
      p['We2'], p['be2'].reshape(N_EXPERTS, 1, D_MODEL))


def _cln2_body(o1_ref, o2_ref, w1_ref, w2_ref, xres_ref, g_ref, b_ref,
               mask_ref, out_ref):
    y = w1_ref[...] * o1_ref[...] + w2_ref[...] * o2_ref[...] + xres_ref[...]
    m = jnp.mean(y, axis=1, keepdims=True)
    v = jnp.mean((y - m) ** 2, axis=1, keepdims=True)
    x2 = (y - m) / jnp.sqrt(v + 1e-5) * g_ref[...] + b_ref[...]
    out_ref[...] = x2 * mask_ref[...]


def _combine_ln2(o1, o2, w1, w2, xres, p, mask):
    blk = 512
    return pl.pallas_call(
        _cln2_body,
        grid=(N_TOK // blk,),
        in_specs=[
            pl.BlockSpec((blk, D_MODEL), lambda i: (i, 0)),
            pl.BlockSpec((blk, D_MODEL), lambda i: (i, 0)),
            pl.BlockSpec((blk, 1), lambda i: (i, 0)),
            pl.BlockSpec((blk, 1), lambda i: (i, 0)),
            pl.BlockSpec((blk, D_MODEL), lambda i: (i, 0)),
            pl.BlockSpec((1, D_MODEL), lambda i: (0, 0)),
            pl.BlockSpec((1, D_MODEL), lambda i: (0, 0)),
            pl.BlockSpec((blk, 1), lambda i: (i, 0)),
        ],
        out_specs=pl.BlockSpec((blk, D_MODEL), lambda i: (i, 0)),
        out_shape=jax.ShapeDtypeStruct((N_TOK, D_MODEL), jnp.float32),
    )(o1, o2, w1, w2, xres, p['g2'].reshape(1, -1), p['b2'].reshape(1, -1),
      mask)


def kernel(frac, params, src):
    p = params
    frac2d = frac.reshape(N_TOK, 1)
    fracr = frac.reshape(N_TOK // ATT_ROWS, 1, ATT_ROWS)
    src2d = src.reshape(N_TOK, 1).astype(jnp.int32)
    wqkv = jnp.concatenate([p['Wq'], p['Wk'], p['Wv']], axis=1)
    bqkv = jnp.concatenate([p['bq'], p['bk'], p['bv']]).reshape(1, -1)
    ones = jnp.ones((N_TOK, 1), jnp.float32)
    finalmask = (frac2d != 0.0).astype(jnp.float32)

    x = _embed(src2d, frac2d, p)
    for layer in range(3):
        ctx = _attn(x, frac2d, fracr, p, wqkv, bqkv)
        x1, i1, i2, w1, w2 = _post(ctx, x, p, p['Wo'], p['bo'].reshape(1, -1))
        p1, p2, meta = _positions(i1, i2)
        p1f = p1.reshape(N_TOK)
        p2f = p2.reshape(N_TOK)
        xs = _sc_dispatch(x1, p1f, p2f)
        ys = _gmm(meta, xs, p)
        o1, o2 = _sc_gather2(ys, p1f, p2f)
        x = _combine_ln2(o1, o2, w1, w2, x1, p,
                         finalmask if layer == 2 else ones)
    return x.reshape(B, T, D_MODEL)


# speedup vs baseline: 1.7736x; 1.0113x over previous
import functools

import jax
import jax.numpy as jnp
import numpy as np
from jax.experimental import pallas as pl
from jax.experimental.pallas import tpu as pltpu
from jax.experimental.pallas import tpu_sc as plsc

D_MODEL = 1024
N_HEADS = 16
HEAD_DIM = 64
N_EXPERTS = 8
D_FF = 2048
N_BASIS = 10
DEGREE = 3
VOCAB = 120
FEAT = 200
B = 512
T = 8
N_TOK = B * T
N_SLOT = 2 * N_TOK
GBLK = 256
NGB = N_SLOT // GBLK
NSTEP = NGB + N_EXPERTS

_NC, _NS = 2, 16
_NW = _NC * _NS

_base = np.linspace(0.0, 1.0, N_BASIS + DEGREE + 1 - 2 * DEGREE)
_KNOTS = np.concatenate(
    [np.repeat(_base[:1], DEGREE), _base, np.repeat(_base[-1:], DEGREE)]
).astype(np.float64)


def _bspline_basis(f):
    nk = _KNOTS.shape[0]
    Bp = [
        jnp.where((f >= float(_KNOTS[i])) & (f < float(_KNOTS[i + 1])), 1.0, 0.0)
        for i in range(nk - 1)
    ]
    for d in range(1, DEGREE + 1):
        Bc = []
        for i in range(nk - d - 1):
            den1 = float(_KNOTS[i + d] - _KNOTS[i])
            den2 = float(_KNOTS[i + d + 1] - _KNOTS[i + 1])
            t = jnp.zeros_like(f)
            if den1 != 0.0:
                t = t + (f - float(_KNOTS[i])) / den1 * Bp[i]
            if den2 != 0.0:
                t = t + (float(_KNOTS[i + d + 1]) - f) / den2 * Bp[i + 1]
            Bc.append(t)
        Bp = Bc
    return jnp.concatenate(Bp, axis=1)


def _embed_body(src_ref, frac_ref, cbfv_ref, we_ref, be_ref, wpe_ref, bpe_ref,
                wple_ref, bple_ref, sc_ref, out_ref):
    src = src_ref[...]
    rows = src.shape[0]
    oh = (src == jax.lax.broadcasted_iota(jnp.int32, (rows, VOCAB), 1)).astype(
        jnp.float32)
    feats = jnp.dot(oh, cbfv_ref[...], preferred_element_type=jnp.float32)
    x = jnp.dot(feats, we_ref[...], preferred_element_type=jnp.float32)
    x = x + be_ref[...]
    emb_scaler = sc_ref[0, 0]
    pos_scaler = sc_ref[0, 1]
    pos_scaler_log = sc_ref[0, 2]
    x = x * jnp.exp2(emb_scaler)
    pe_scaler = jnp.exp2((1.0 - pos_scaler) ** 2)
    ple_scaler = jnp.exp2((1.0 - pos_scaler_log) ** 2)

    f = jnp.clip(frac_ref[...], 1e-9, 1.0)
    basis = _bspline_basis(f)
    pe = (jnp.dot(basis, wpe_ref[...], preferred_element_type=jnp.float32)
          + bpe_ref[...]) * pe_scaler
    f2 = jnp.clip(0.0025 * jnp.log2(f) ** 2, 0.0, 1.0)
    basis2 = _bspline_basis(f2)
    ple = (jnp.dot(basis2, wple_ref[...], preferred_element_type=jnp.float32)
           + bple_ref[...]) * ple_scaler
    out_ref[...] = x + jnp.concatenate([pe, ple], axis=1)


def _embed(src2d, frac2d, p):
    blk = 512
    grid = (N_TOK // blk,)
    scalars = jnp.stack([p['emb_scaler'], p['pos_scaler'],
                         p['pos_scaler_log']]).reshape(1, 3)
    half = D_MODEL // 2
    return pl.pallas_call(
        _embed_body,
        grid=grid,
        in_specs=[
            pl.BlockSpec((blk, 1), lambda i: (i, 0)),
            pl.BlockSpec((blk, 1), lambda i: (i, 0)),
            pl.BlockSpec((VOCAB, FEAT), lambda i: (0, 0)),
            pl.BlockSpec((FEAT, D_MODEL), lambda i: (0, 0)),
            pl.BlockSpec((1, D_MODEL), lambda i: (0, 0)),
            pl.BlockSpec((N_BASIS, half), lambda i: (0, 0)),
            pl.BlockSpec((1, half), lambda i: (0, 0)),
            pl.BlockSpec((N_BASIS, half), lambda i: (0, 0)),
            pl.BlockSpec((1, half), lambda i: (0, 0)),
            pl.BlockSpec((1, 3), lambda i: (0, 0)),
        ],
        out_specs=pl.BlockSpec((blk, D_MODEL), lambda i: (i, 0)),
        out_shape=jax.ShapeDtypeStruct((N_TOK, D_MODEL), jnp.float32),
    )(src2d, frac2d, p['cbfv'], p['We'], p['be'].reshape(1, -1),
      p['W_pe'], p['b_pe'].reshape(1, -1), p['W_ple'], p['b_ple'].reshape(1, -1),
      scalars)


ATT_ROWS = 256


def _attn_body(x_ref, frac_ref, fracr_ref, wqkv_ref, bqkv_ref, alpha_ref,
               out_ref):
    x = x_ref[...].astype(jnp.bfloat16)
    qkv = jnp.dot(x, wqkv_ref[...], preferred_element_type=jnp.float32)
    qkv = qkv + bqkv_ref[...]
    qkvb = qkv.astype(jnp.bfloat16)
    fcol = frac_ref[...]
    frow = fracr_ref[0]
    alpha = alpha_ref[0, 0]
    R = ATT_ROWS
    diff = fcol - frow
    bias = alpha * (jnp.log1p(jnp.abs(diff)) * jnp.sign(diff))
    ii = jax.lax.broadcasted_iota(jnp.int32, (R, R), 0)
    jj = jax.lax.broadcasted_iota(jnp.int32, (R, R), 1)
    same = (ii // T) == (jj // T)
    keyok = frow != 0.0
    valid = same & keyok
    scale = HEAD_DIM ** -0.5
    for h in range(N_HEADS):
        q = qkvb[:, h * HEAD_DIM:(h + 1) * HEAD_DIM]
        k = qkvb[:, D_MODEL + h * HEAD_DIM:D_MODEL + (h + 1) * HEAD_DIM]
        v = qkvb[:, 2 * D_MODEL + h * HEAD_DIM:2 * D_MODEL + (h + 1) * HEAD_DIM]
        s = jax.lax.dot_general(q, k, (((1,), (1,)), ((), ())),
                                preferred_element_type=jnp.float32) * scale
        s = jnp.where(valid, s + bias, -1e30)
        m = jnp.max(s, axis=1, keepdims=True)
        e = jnp.exp(s - m)
        pr = e * (1.0 / jnp.sum(e, axis=1, keepdims=True))
        ctx = jnp.dot(pr.astype(jnp.bfloat16), v,
                      preferred_element_type=jnp.float32)
        out_ref[:, h * HEAD_DIM:(h + 1) * HEAD_DIM] = ctx


def _attn(x, frac2d, fracr, p, wqkv, bqkv):
    grid = (N_TOK // ATT_ROWS,)
    return pl.pallas_call(
        _attn_body,
        grid=grid,
        in_specs=[
            pl.BlockSpec((ATT_ROWS, D_MODEL), lambda i: (i, 0)),
            pl.BlockSpec((ATT_ROWS, 1), lambda i: (i, 0)),
            pl.BlockSpec((1, 1, ATT_ROWS), lambda i: (i, 0, 0)),
            pl.BlockSpec((D_MODEL, 3 * D_MODEL), lambda i: (0, 0)),
            pl.BlockSpec((1, 3 * D_MODEL), lambda i: (0, 0)),
            pl.BlockSpec((1, 1), lambda i: (0, 0)),
        ],
        out_specs=pl.BlockSpec((ATT_ROWS, D_MODEL), lambda i: (i, 0)),
        out_shape=jax.ShapeDtypeStruct((N_TOK, D_MODEL), jnp.float32),
    )(x, frac2d, fracr, wqkv, bqkv, p['alpha'].reshape(1, 1))


def _post_body(ctx_ref, xin_ref, wo_ref, bo_ref, g1_ref, b1_ref, wg_ref,
               bg_ref, x1_ref, i1_ref, i2_ref, w1_ref, w2_ref):
    y = jnp.dot(ctx_ref[...].astype(jnp.bfloat16), wo_ref[...],
                preferred_element_type=jnp.float32)
    y = y + bo_ref[...] + xin_ref[...]
    m = jnp.mean(y, axis=1, keepdims=True)
    v = jnp.mean((y - m) ** 2, axis=1, keepdims=True)
    x1 = (y - m) / jnp.sqrt(v + 1e-5) * g1_ref[...] + b1_ref[...]
    x1_ref[...] = x1
    logits = jnp.dot(x1.astype(jnp.bfloat16), wg_ref[...],
                     preferred_element_type=jnp.float32)
    logits = logits + bg_ref[...]
    lm = jnp.max(logits, axis=1, keepdims=True)
    le = jnp.exp(logits - lm)
    probs = le / jnp.sum(le, axis=1, keepdims=True)
    rows = probs.shape[0]
    lane = jax.lax.broadcasted_iota(jnp.int32, (rows, N_EXPERTS), 1)
    w1 = jnp.max(probs, axis=1, keepdims=True)
    i1 = jnp.min(jnp.where(probs == w1, lane, N_EXPERTS), axis=1, keepdims=True)
    probs2 = jnp.where(lane == i1, -1.0, probs)
    w2 = jnp.max(probs2, axis=1, keepdims=True)
    i2 = jnp.min(jnp.where(probs2 == w2, lane, N_EXPERTS), axis=1, keepdims=True)
    i1_ref[...] = i1
    i2_ref[...] = i2
    w1_ref[...] = w1
    w2_ref[...] = w2


def _post(ctx, xin, p, wo, bo, wg):
    blk = 512
    grid = (N_TOK // blk,)
    return pl.pallas_call(
        _post_body,
        grid=grid,
        in_specs=[
            pl.BlockSpec((blk, D_MODEL), lambda i: (i, 0)),
            pl.BlockSpec((blk, D_MODEL), lambda i: (i, 0)),
            pl.BlockSpec((D_MODEL, D_MODEL), lambda i: (0, 0)),
            pl.BlockSpec((1, D_MODEL), lambda i: (0, 0)),
            pl.BlockSpec((1, D_MODEL), lambda i: (0, 0)),
            pl.BlockSpec((1, D_MODEL), lambda i: (0, 0)),
            pl.BlockSpec((D_MODEL, N_EXPERTS), lambda i: (0, 0)),
            pl.BlockSpec((1, N_EXPERTS), lambda i: (0, 0)),
        ],
        out_specs=[
            pl.BlockSpec((blk, D_MODEL), lambda i: (i, 0)),
            pl.BlockSpec((blk, 1), lambda i: (i, 0)),
            pl.BlockSpec((blk, 1), lambda i: (i, 0)),
            pl.BlockSpec((blk, 1), lambda i: (i, 0)),
            pl.BlockSpec((blk, 1), lambda i: (i, 0)),
        ],
        out_shape=[
            jax.ShapeDtypeStruct((N_TOK, D_MODEL), jnp.float32),
            jax.ShapeDtypeStruct((N_TOK, 1), jnp.int32),
            jax.ShapeDtypeStruct((N_TOK, 1), jnp.int32),
            jax.ShapeDtypeStruct((N_TOK, 1), jnp.float32),
            jax.ShapeDtypeStruct((N_TOK, 1), jnp.float32),
        ],
    )(ctx, xin, wo, bo, p['g1'].reshape(1, -1), p['b1'].reshape(1, -1),
      wg, p['bg'].reshape(1, -1))


def _cumsum_rows(a):
    n, w = a.shape
    sh = 1
    while sh < n:
        a = a + jnp.concatenate(
            [jnp.zeros((sh, w), a.dtype), a[:-sh]], axis=0)
        sh *= 2
    return a


def _pos_body(i1_ref, i2_ref, p1_ref, p2_ref, meta_ref):
    lane = jax.lax.broadcasted_iota(jnp.int32, (N_TOK, N_EXPERTS), 1)
    h1 = (i1_ref[...] == lane).astype(jnp.int32)
    h2 = (i2_ref[...] == lane).astype(jnp.int32)
    c1 = _cumsum_rows(h1)
    c2 = _cumsum_rows(h2)
    cnt1 = c1[N_TOK - 1:N_TOK, :]
    cnt2 = c2[N_TOK - 1:N_TOK, :]
    counts = cnt1 + cnt2
    lane8 = jax.lax.broadcasted_iota(jnp.int32, (1, N_EXPERTS), 1)

    offs_row = jnp.zeros((1, N_EXPERTS), jnp.int32)
    off = jnp.int32(0)
    off_e = []
    cnt_e = []
    cnt1_e = []
    for e in range(N_EXPERTS):
        ce = jnp.sum(jnp.where(lane8 == e, counts, 0))
        c1e = jnp.sum(jnp.where(lane8 == e, cnt1, 0))
        off_e.append(off)
        cnt_e.append(ce)
        cnt1_e.append(c1e)
        offs_row = offs_row + jnp.where(lane8 == e, off, 0)
        off = off + ce

    cnt1_row = cnt1
    p1_ref[...] = jnp.sum(h1 * (offs_row + c1 - h1), axis=1, keepdims=True)
    p2_ref[...] = jnp.sum(h2 * (offs_row + cnt1_row + c2 - h2), axis=1,
                          keepdims=True)

    lane_s = jax.lax.broadcasted_iota(jnp.int32, (1, NSTEP), 1)
    rb_row = jnp.zeros((1, NSTEP), jnp.int32)
    e_row = jnp.zeros((1, NSTEP), jnp.int32)
    st_row = jnp.zeros((1, NSTEP), jnp.int32)
    en_row = jnp.zeros((1, NSTEP), jnp.int32)
    any_row = jnp.zeros((1, NSTEP), jnp.int32)
    cum = jnp.int32(0)
    for e in range(N_EXPERTS):
        start = off_e[e]
        end = off_e[e] + cnt_e[e]
        nonempty = cnt_e[e] > 0
        fb = start // GBLK
        lb = jnp.where(nonempty, (end - 1) // GBLK, 0)
        nb = jnp.where(nonempty, lb - fb + 1, 0)
        active = (lane_s >= cum) & (lane_s < cum + nb)
        rb_here = fb + (lane_s - cum)
        rb_row = rb_row + jnp.where(active, rb_here, 0)
        e_row = e_row + jnp.where(active, e, 0)
        st_row = st_row + jnp.where(active, jnp.maximum(start, rb_here * GBLK), 0)
        en_row = en_row + jnp.where(active, jnp.minimum(end, (rb_here + 1) * GBLK), 0)
        any_row = any_row + active.astype(jnp.int32)
        cum = cum + nb
    rb_row = jnp.where(any_row > 0, rb_row, NGB - 1)
    prev = jnp.concatenate(
        [jnp.full((1, 1), -1, jnp.int32), rb_row[:, :NSTEP - 1]], axis=1)
    first_row = (rb_row != prev).astype(jnp.int32)
    meta_ref[...] = jnp.concatenate(
        [rb_row, e_row, first_row, st_row, en_row], axis=0)


def _positions(i1, i2):
    return pl.pallas_call(
        _pos_body,
        grid=(1,),
        in_specs=[
            pl.BlockSpec((N_TOK, 1), lambda i: (0, 0)),
            pl.BlockSpec((N_TOK, 1), lambda i: (0, 0)),
        ],
        out_specs=[
            pl.BlockSpec((N_TOK, 1), lambda i: (0, 0)),
            pl.BlockSpec((N_TOK, 1), lambda i: (0, 0)),
            pl.BlockSpec((5, NSTEP), lambda i: (0, 0)),
        ],
        out_shape=[
            jax.ShapeDtypeStruct((N_TOK, 1), jnp.int32),
            jax.ShapeDtypeStruct((N_TOK, 1), jnp.int32),
            jax.ShapeDtypeStruct((5, NSTEP), jnp.int32),
        ],
    )(i1, i2)


_SC_CH = 64


def _sc_mesh():
    return plsc.VectorSubcoreMesh(core_axis_name="c", subcore_axis_name="s",
                                  num_cores=_NC, num_subcores=_NS)


def _sc_dispatch(x1, p1f, p2f):
    tok_per_w = N_TOK // _NW

    def body(x_hbm, p1_hbm, p2_hbm, xs_hbm, idx_v, rows_v, sem):
        wid = jax.lax.axis_index("s") * _NC + jax.lax.axis_index("c")
        base0 = wid * tok_per_w
        for c in range(tok_per_w // _SC_CH):
            base = base0 + c * _SC_CH
            pltpu.sync_copy(x_hbm.at[pl.ds(base, _SC_CH)], rows_v)
            pltpu.sync_copy(p1_hbm.at[pl.ds(base, _SC_CH)], idx_v)
            pltpu.async_copy(rows_v, xs_hbm.at[idx_v], sem).wait()
            pltpu.sync_copy(p2_hbm.at[pl.ds(base, _SC_CH)], idx_v)
            pltpu.async_copy(rows_v, xs_hbm.at[idx_v], sem).wait()

    f = pl.kernel(
        body,
        out_type=jax.ShapeDtypeStruct((N_SLOT, D_MODEL), jnp.float32),
        mesh=_sc_mesh(),
        scratch_types=[
            pltpu.VMEM((_SC_CH,), jnp.int32),
            pltpu.VMEM((_SC_CH, D_MODEL), jnp.float32),
            pltpu.SemaphoreType.DMA,
        ],
    )
    return f(x1, p1f, p2f)


def _sc_gather2(ys, p1f, p2f):
    tok_per_w = N_TOK // _NW

    def body(ys_hbm, p1_hbm, p2_hbm, o1_hbm, o2_hbm, idx_v, rows_v, sem):
        wid = jax.lax.axis_index("s") * _NC + jax.lax.axis_index("c")
        base0 = wid * tok_per_w
        for c in range(tok_per_w // _SC_CH):
            base = base0 + c * _SC_CH
            pltpu.sync_copy(p1_hbm.at[pl.ds(base, _SC_CH)], idx_v)
            pltpu.async_copy(ys_hbm.at[idx_v], rows_v, sem).wait()
            pltpu.sync_copy(rows_v, o1_hbm.at[pl.ds(base, _SC_CH)])
            pltpu.sync_copy(p2_hbm.at[pl.ds(base, _SC_CH)], idx_v)
            pltpu.async_copy(ys_hbm.at[idx_v], rows_v, sem).wait()
            pltpu.sync_copy(rows_v, o2_hbm.at[pl.ds(base, _SC_CH)])

    f = pl.kernel(
        body,
        out_type=[
            jax.ShapeDtypeStruct((N_TOK, D_MODEL), jnp.float32),
            jax.ShapeDtypeStruct((N_TOK, D_MODEL), jnp.float32),
        ],
        mesh=_sc_mesh(),
        scratch_types=[
            pltpu.VMEM((_SC_CH,), jnp.int32),
            pltpu.VMEM((_SC_CH, D_MODEL), jnp.float32),
            pltpu.SemaphoreType.DMA,
        ],
    )
    return f(ys, p1f, p2f)


def _gmm_body(meta_ref, xs_ref, we1_ref, be1_ref, we2_ref, be2_ref, ys_ref):
    s = pl.program_id(0)
    rb = meta_ref[0, s]
    first = meta_ref[2, s]
    start = meta_ref[3, s]
    end = meta_ref[4, s]
    x = xs_ref[...].astype(jnp.bfloat16)
    h = jnp.dot(x, we1_ref[0], preferred_element_type=jnp.float32)
    h = jnp.maximum(h + be1_ref[0], 0.0)
    y = jnp.dot(h.astype(jnp.bfloat16), we2_ref[0],
                preferred_element_type=jnp.float32)
    y = y + be2_ref[0]
    gi = rb * GBLK + jax.lax.broadcasted_iota(jnp.int32, (GBLK, 1), 0)
    rowmask = (gi >= start) & (gi < end)
    contrib = jnp.where(rowmask, y, 0.0)

    @pl.when(first == 1)
    def _init():
        ys_ref[...] = contrib

    @pl.when(first == 0)
    def _acc():
        ys_ref[...] = ys_ref[...] + contrib


def _gmm(meta, xs, we1b, we2b, p):
    grid_spec = pltpu.PrefetchScalarGridSpec(
        num_scalar_prefetch=1,
        grid=(NSTEP,),
        in_specs=[
            pl.BlockSpec((GBLK, D_MODEL), lambda s, m: (m[0, s], 0)),
            pl.BlockSpec((1, D_MODEL, D_FF), lambda s, m: (m[1, s], 0, 0)),
            pl.BlockSpec((1, 1, D_FF), lambda s, m: (m[1, s], 0, 0)),
            pl.BlockSpec((1, D_FF, D_MODEL), lambda s, m: (m[1, s], 0, 0)),
            pl.BlockSpec((1, 1, D_MODEL), lambda s, m: (m[1, s], 0, 0)),
        ],
        out_specs=pl.BlockSpec((GBLK, D_MODEL), lambda s, m: (m[0, s], 0)),
    )
    return pl.pallas_call(
        _gmm_body,
        grid_spec=grid_spec,
        out_shape=jax.ShapeDtypeStruct((N_SLOT, D_MODEL), jnp.float32),
    )(meta, xs, we1b, p['be1'].reshape(N_EXPERTS, 1, D_FF),
      we2b, p['be2'].reshape(N_EXPERTS, 1, D_MODEL))


def _cln2_body(o1_ref, o2_ref, w1_ref, w2_ref, xres_ref, g_ref, b_ref,
               mask_ref, out_ref):
    y = w1_ref[...] * o1_ref[...] + w2_ref[...] * o2_ref[...] + xres_ref[...]
    m = jnp.mean(y, axis=1, keepdims=True)
    v = jnp.mean((y - m) ** 2, axis=1, keepdims=True)
    x2 = (y - m) / jnp.sqrt(v + 1e-5) * g_ref[...] + b_ref[...]
    out_ref[...] = x2 * mask_ref[...]


def _combine_ln2(o1, o2, w1, w2, xres, p, mask):
    blk = 512
    return pl.pallas_call(
        _cln2_body,
        grid=(N_TOK // blk,),
        in_specs=[
            pl.BlockSpec((blk, D_MODEL), lambda i: (i, 0)),
            pl.BlockSpec((blk, D_MODEL), lambda i: (i, 0)),
            pl.BlockSpec((blk, 1), lambda i: (i, 0)),
            pl.BlockSpec((blk, 1), lambda i: (i, 0)),
            pl.BlockSpec((blk, D_MODEL), lambda i: (i, 0)),
            pl.BlockSpec((1, D_MODEL), lambda i: (0, 0)),
            pl.BlockSpec((1, D_MODEL), lambda i: (0, 0)),
            pl.BlockSpec((blk, 1), lambda i: (i, 0)),
        ],
        out_specs=pl.BlockSpec((blk, D_MODEL), lambda i: (i, 0)),
        out_shape=jax.ShapeDtypeStruct((N_TOK, D_MODEL), jnp.float32),
    )(o1, o2, w1, w2, xres, p['g2'].reshape(1, -1), p['b2'].reshape(1, -1),
      mask)


def kernel(frac, params, src):
    p = params
    frac2d = frac.reshape(N_TOK, 1)
    fracr = frac.reshape(N_TOK // ATT_ROWS, 1, ATT_ROWS)
    src2d = src.reshape(N_TOK, 1).astype(jnp.int32)
    wqkv = jnp.concatenate([p['Wq'], p['Wk'], p['Wv']], axis=1)
    bqkv = jnp.concatenate([p['bq'], p['bk'], p['bv']]).reshape(1, -1)
    ones = jnp.ones((N_TOK, 1), jnp.float32)
    finalmask = (frac2d != 0.0).astype(jnp.float32)
    we1b = p['We1'].astype(jnp.bfloat16)
    we2b = p['We2'].astype(jnp.bfloat16)
    wqkv = wqkv.astype(jnp.bfloat16)
    wob = p['Wo'].astype(jnp.bfloat16)
    wgb = p['Wg'].astype(jnp.bfloat16)

    x = _embed(src2d, frac2d, p)
    for layer in range(3):
        ctx = _attn(x, frac2d, fracr, p, wqkv, bqkv)
        x1, i1, i2, w1, w2 = _post(ctx, x, p, wob, p['bo'].reshape(1, -1),
                                   wgb)
        p1, p2, meta = _positions(i1, i2)
        p1f = p1.reshape(N_TOK)
        p2f = p2.reshape(N_TOK)
        xs = _sc_dispatch(x1, p1f, p2f)
        ys = _gmm(meta, xs, we1b, we2b, p)
        o1, o2 = _sc_gather2(ys, p1f, p2f)
        x = _combine_ln2(o1, o2, w1, w2, x1, p,
                         finalmask if layer == 2 else ones)
    return x.reshape(B, T, D_MODEL)


# speedup vs baseline: 1.8347x; 1.0344x over previous
import functools

import jax
import jax.numpy as jnp
import numpy as np
from jax.experimental import pallas as pl
from jax.experimental.pallas import tpu as pltpu
from jax.experimental.pallas import tpu_sc as plsc

D_MODEL = 1024
N_HEADS = 16
HEAD_DIM = 64
N_EXPERTS = 8
D_FF = 2048
N_BASIS = 10
DEGREE = 3
VOCAB = 120
FEAT = 200
B = 512
T = 8
N_TOK = B * T
N_SLOT = 2 * N_TOK
GBLK = 256
NGB = N_SLOT // GBLK
NSTEP = NGB + N_EXPERTS

_NC, _NS = 2, 16
_NW = _NC * _NS

_base = np.linspace(0.0, 1.0, N_BASIS + DEGREE + 1 - 2 * DEGREE)
_KNOTS = np.concatenate(
    [np.repeat(_base[:1], DEGREE), _base, np.repeat(_base[-1:], DEGREE)]
).astype(np.float64)


def _bspline_basis(f):
    nk = _KNOTS.shape[0]
    Bp = [
        jnp.where((f >= float(_KNOTS[i])) & (f < float(_KNOTS[i + 1])), 1.0, 0.0)
        for i in range(nk - 1)
    ]
    for d in range(1, DEGREE + 1):
        Bc = []
        for i in range(nk - d - 1):
            den1 = float(_KNOTS[i + d] - _KNOTS[i])
            den2 = float(_KNOTS[i + d + 1] - _KNOTS[i + 1])
            t = jnp.zeros_like(f)
            if den1 != 0.0:
                t = t + (f - float(_KNOTS[i])) / den1 * Bp[i]
            if den2 != 0.0:
                t = t + (float(_KNOTS[i + d + 1]) - f) / den2 * Bp[i + 1]
            Bc.append(t)
        Bp = Bc
    return jnp.concatenate(Bp, axis=1)


ATT_ROWS = 256


def _embed_rows(src, frac, cbfv_ref, we_ref, be_ref, wpe_ref, bpe_ref,
                wple_ref, bple_ref, sc_ref):
    rows = src.shape[0]
    oh = (src == jax.lax.broadcasted_iota(jnp.int32, (rows, VOCAB), 1)).astype(
        jnp.float32)
    feats = jnp.dot(oh, cbfv_ref[...], preferred_element_type=jnp.float32)
    x = jnp.dot(feats, we_ref[...], preferred_element_type=jnp.float32)
    x = x + be_ref[...]
    emb_scaler = sc_ref[0, 0]
    pos_scaler = sc_ref[0, 1]
    pos_scaler_log = sc_ref[0, 2]
    x = x * jnp.exp2(emb_scaler)
    pe_scaler = jnp.exp2((1.0 - pos_scaler) ** 2)
    ple_scaler = jnp.exp2((1.0 - pos_scaler_log) ** 2)

    f = jnp.clip(frac, 1e-9, 1.0)
    basis = _bspline_basis(f)
    pe = (jnp.dot(basis, wpe_ref[...], preferred_element_type=jnp.float32)
          + bpe_ref[...]) * pe_scaler
    f2 = jnp.clip(0.0025 * jnp.log2(f) ** 2, 0.0, 1.0)
    basis2 = _bspline_basis(f2)
    ple = (jnp.dot(basis2, wple_ref[...], preferred_element_type=jnp.float32)
           + bple_ref[...]) * ple_scaler
    return x + jnp.concatenate([pe, ple], axis=1)


def _attn_core(x, frac_ref, fracr_ref, wqkv_ref, bqkv_ref, alpha_ref, out_ref):
    qkv = jnp.dot(x.astype(jnp.bfloat16), wqkv_ref[...],
                  preferred_element_type=jnp.float32)
    qkv = qkv + bqkv_ref[...]
    qkvb = qkv.astype(jnp.bfloat16)
    fcol = frac_ref[...]
    frow = fracr_ref[0]
    alpha = alpha_ref[0, 0]
    R = ATT_ROWS
    diff = fcol - frow
    bias = alpha * (jnp.log1p(jnp.abs(diff)) * jnp.sign(diff))
    ii = jax.lax.broadcasted_iota(jnp.int32, (R, R), 0)
    jj = jax.lax.broadcasted_iota(jnp.int32, (R, R), 1)
    same = (ii // T) == (jj // T)
    keyok = frow != 0.0
    valid = same & keyok
    scale = HEAD_DIM ** -0.5
    for h in range(N_HEADS):
        q = qkvb[:, h * HEAD_DIM:(h + 1) * HEAD_DIM]
        k = qkvb[:, D_MODEL + h * HEAD_DIM:D_MODEL + (h + 1) * HEAD_DIM]
        v = qkvb[:, 2 * D_MODEL + h * HEAD_DIM:2 * D_MODEL + (h + 1) * HEAD_DIM]
        s = jax.lax.dot_general(q, k, (((1,), (1,)), ((), ())),
                                preferred_element_type=jnp.float32) * scale
        s = jnp.where(valid, s + bias, -1e30)
        m = jnp.max(s, axis=1, keepdims=True)
        e = jnp.exp(s - m)
        pr = e * (1.0 / jnp.sum(e, axis=1, keepdims=True))
        ctx = jnp.dot(pr.astype(jnp.bfloat16), v,
                      preferred_element_type=jnp.float32)
        out_ref[:, h * HEAD_DIM:(h + 1) * HEAD_DIM] = ctx


def _attn_first_body(src_ref, frac_ref, fracr_ref, cbfv_ref, we_ref, be_ref,
                     wpe_ref, bpe_ref, wple_ref, bple_ref, sc_ref,
                     wqkv_ref, bqkv_ref, alpha_ref, ctx_ref, x0_ref):
    x0 = _embed_rows(src_ref[...], frac_ref[...], cbfv_ref, we_ref, be_ref,
                     wpe_ref, bpe_ref, wple_ref, bple_ref, sc_ref)
    x0_ref[...] = x0
    _attn_core(x0, frac_ref, fracr_ref, wqkv_ref, bqkv_ref, alpha_ref, ctx_ref)


def _attn_first(src2d, frac2d, fracr, p, wqkv, bqkv):
    grid = (N_TOK // ATT_ROWS,)
    half = D_MODEL // 2
    scalars = jnp.stack([p['emb_scaler'], p['pos_scaler'],
                         p['pos_scaler_log']]).reshape(1, 3)
    full = lambda *shape: pl.BlockSpec(shape, lambda i: (0,) * len(shape))
    return pl.pallas_call(
        _attn_first_body,
        grid=grid,
        in_specs=[
            pl.BlockSpec((ATT_ROWS, 1), lambda i: (i, 0)),
            pl.BlockSpec((ATT_ROWS, 1), lambda i: (i, 0)),
            pl.BlockSpec((1, 1, ATT_ROWS), lambda i: (i, 0, 0)),
            full(VOCAB, FEAT),
            full(FEAT, D_MODEL),
            full(1, D_MODEL),
            full(N_BASIS, half),
            full(1, half),
            full(N_BASIS, half),
            full(1, half),
            full(1, 3),
            full(D_MODEL, 3 * D_MODEL),
            full(1, 3 * D_MODEL),
            full(1, 1),
        ],
        out_specs=[
            pl.BlockSpec((ATT_ROWS, D_MODEL), lambda i: (i, 0)),
            pl.BlockSpec((ATT_ROWS, D_MODEL), lambda i: (i, 0)),
        ],
        out_shape=[
            jax.ShapeDtypeStruct((N_TOK, D_MODEL), jnp.float32),
            jax.ShapeDtypeStruct((N_TOK, D_MODEL), jnp.float32),
        ],
    )(src2d, frac2d, fracr, p['cbfv'], p['We'], p['be'].reshape(1, -1),
      p['W_pe'], p['b_pe'].reshape(1, -1), p['W_ple'],
      p['b_ple'].reshape(1, -1), scalars, wqkv, bqkv,
      p['alpha'].reshape(1, 1))


def _attn_mid_body(o1_ref, o2_ref, w1_ref, w2_ref, xres_ref, g2_ref, b2_ref,
                   frac_ref, fracr_ref, wqkv_ref, bqkv_ref, alpha_ref,
                   ctx_ref, x_ref):
    y = (w1_ref[...] * o1_ref[...] + w2_ref[...] * o2_ref[...]
         + xres_ref[...])
    m = jnp.mean(y, axis=1, keepdims=True)
    v = jnp.mean((y - m) ** 2, axis=1, keepdims=True)
    x = (y - m) / jnp.sqrt(v + 1e-5) * g2_ref[...] + b2_ref[...]
    x_ref[...] = x
    _attn_core(x, frac_ref, fracr_ref, wqkv_ref, bqkv_ref, alpha_ref, ctx_ref)


def _attn_mid(o1, o2, w1, w2, xres, frac2d, fracr, p, wqkv, bqkv):
    grid = (N_TOK // ATT_ROWS,)
    full = lambda *shape: pl.BlockSpec(shape, lambda i: (0,) * len(shape))
    return pl.pallas_call(
        _attn_mid_body,
        grid=grid,
        in_specs=[
            pl.BlockSpec((ATT_ROWS, D_MODEL), lambda i: (i, 0)),
            pl.BlockSpec((ATT_ROWS, D_MODEL), lambda i: (i, 0)),
            pl.BlockSpec((ATT_ROWS, 1), lambda i: (i, 0)),
            pl.BlockSpec((ATT_ROWS, 1), lambda i: (i, 0)),
            pl.BlockSpec((ATT_ROWS, D_MODEL), lambda i: (i, 0)),
            full(1, D_MODEL),
            full(1, D_MODEL),
            pl.BlockSpec((ATT_ROWS, 1), lambda i: (i, 0)),
            pl.BlockSpec((1, 1, ATT_ROWS), lambda i: (i, 0, 0)),
            full(D_MODEL, 3 * D_MODEL),
            full(1, 3 * D_MODEL),
            full(1, 1),
        ],
        out_specs=[
            pl.BlockSpec((ATT_ROWS, D_MODEL), lambda i: (i, 0)),
            pl.BlockSpec((ATT_ROWS, D_MODEL), lambda i: (i, 0)),
        ],
        out_shape=[
            jax.ShapeDtypeStruct((N_TOK, D_MODEL), jnp.float32),
            jax.ShapeDtypeStruct((N_TOK, D_MODEL), jnp.float32),
        ],
    )(o1, o2, w1, w2, xres, p['g2'].reshape(1, -1), p['b2'].reshape(1, -1),
      frac2d, fracr, wqkv, bqkv, p['alpha'].reshape(1, 1))


def _post_body(ctx_ref, xin_ref, wo_ref, bo_ref, g1_ref, b1_ref, wg_ref,
               bg_ref, x1_ref, i1_ref, i2_ref, w1_ref, w2_ref):
    y = jnp.dot(ctx_ref[...].astype(jnp.bfloat16), wo_ref[...],
                preferred_element_type=jnp.float32)
    y = y + bo_ref[...] + xin_ref[...]
    m = jnp.mean(y, axis=1, keepdims=True)
    v = jnp.mean((y - m) ** 2, axis=1, keepdims=True)
    x1 = (y - m) / jnp.sqrt(v + 1e-5) * g1_ref[...] + b1_ref[...]
    x1_ref[...] = x1
    logits = jnp.dot(x1.astype(jnp.bfloat16), wg_ref[...],
                     preferred_element_type=jnp.float32)
    logits = logits + bg_ref[...]
    lm = jnp.max(logits, axis=1, keepdims=True)
    le = jnp.exp(logits - lm)
    probs = le / jnp.sum(le, axis=1, keepdims=True)
    rows = probs.shape[0]
    lane = jax.lax.broadcasted_iota(jnp.int32, (rows, N_EXPERTS), 1)
    w1 = jnp.max(probs, axis=1, keepdims=True)
    i1 = jnp.min(jnp.where(probs == w1, lane, N_EXPERTS), axis=1, keepdims=True)
    probs2 = jnp.where(lane == i1, -1.0, probs)
    w2 = jnp.max(probs2, axis=1, keepdims=True)
    i2 = jnp.min(jnp.where(probs2 == w2, lane, N_EXPERTS), axis=1, keepdims=True)
    i1_ref[...] = i1
    i2_ref[...] = i2
    w1_ref[...] = w1
    w2_ref[...] = w2


def _post(ctx, xin, p, wo, bo, wg):
    blk = 512
    grid = (N_TOK // blk,)
    return pl.pallas_call(
        _post_body,
        grid=grid,
        in_specs=[
            pl.BlockSpec((blk, D_MODEL), lambda i: (i, 0)),
            pl.BlockSpec((blk, D_MODEL), lambda i: (i, 0)),
            pl.BlockSpec((D_MODEL, D_MODEL), lambda i: (0, 0)),
            pl.BlockSpec((1, D_MODEL), lambda i: (0, 0)),
            pl.BlockSpec((1, D_MODEL), lambda i: (0, 0)),
            pl.BlockSpec((1, D_MODEL), lambda i: (0, 0)),
            pl.BlockSpec((D_MODEL, N_EXPERTS), lambda i: (0, 0)),
            pl.BlockSpec((1, N_EXPERTS), lambda i: (0, 0)),
        ],
        out_specs=[
            pl.BlockSpec((blk, D_MODEL), lambda i: (i, 0)),
            pl.BlockSpec((blk, 1), lambda i: (i, 0)),
            pl.BlockSpec((blk, 1), lambda i: (i, 0)),
            pl.BlockSpec((blk, 1), lambda i: (i, 0)),
            pl.BlockSpec((blk, 1), lambda i: (i, 0)),
        ],
        out_shape=[
            jax.ShapeDtypeStruct((N_TOK, D_MODEL), jnp.float32),
            jax.ShapeDtypeStruct((N_TOK, 1), jnp.int32),
            jax.ShapeDtypeStruct((N_TOK, 1), jnp.int32),
            jax.ShapeDtypeStruct((N_TOK, 1), jnp.float32),
            jax.ShapeDtypeStruct((N_TOK, 1), jnp.float32),
        ],
    )(ctx, xin, wo, bo, p['g1'].reshape(1, -1), p['b1'].reshape(1, -1),
      wg, p['bg'].reshape(1, -1))


def _cumsum_rows(a):
    n, w = a.shape
    sh = 1
    while sh < n:
        a = a + jnp.concatenate(
            [jnp.zeros((sh, w), a.dtype), a[:-sh]], axis=0)
        sh *= 2
    return a


def _pos_body(i1_ref, i2_ref, p1_ref, p2_ref, meta_ref):
    lane = jax.lax.broadcasted_iota(jnp.int32, (N_TOK, N_EXPERTS), 1)
    h1 = (i1_ref[...] == lane).astype(jnp.int32)
    h2 = (i2_ref[...] == lane).astype(jnp.int32)
    c1 = _cumsum_rows(h1)
    c2 = _cumsum_rows(h2)
    cnt1 = c1[N_TOK - 1:N_TOK, :]
    cnt2 = c2[N_TOK - 1:N_TOK, :]
    counts = cnt1 + cnt2
    lane8 = jax.lax.broadcasted_iota(jnp.int32, (1, N_EXPERTS), 1)

    offs_row = jnp.zeros((1, N_EXPERTS), jnp.int32)
    off = jnp.int32(0)
    off_e = []
    cnt_e = []
    cnt1_e = []
    for e in range(N_EXPERTS):
        ce = jnp.sum(jnp.where(lane8 == e, counts, 0))
        c1e = jnp.sum(jnp.where(lane8 == e, cnt1, 0))
        off_e.append(off)
        cnt_e.append(ce)
        cnt1_e.append(c1e)
        offs_row = offs_row + jnp.where(lane8 == e, off, 0)
        off = off + ce

    cnt1_row = cnt1
    p1_ref[...] = jnp.sum(h1 * (offs_row + c1 - h1), axis=1, keepdims=True)
    p2_ref[...] = jnp.sum(h2 * (offs_row + cnt1_row + c2 - h2), axis=1,
                          keepdims=True)

    lane_s = jax.lax.broadcasted_iota(jnp.int32, (1, NSTEP), 1)
    rb_row = jnp.zeros((1, NSTEP), jnp.int32)
    e_row = jnp.zeros((1, NSTEP), jnp.int32)
    st_row = jnp.zeros((1, NSTEP), jnp.int32)
    en_row = jnp.zeros((1, NSTEP), jnp.int32)
    any_row = jnp.zeros((1, NSTEP), jnp.int32)
    cum = jnp.int32(0)
    for e in range(N_EXPERTS):
        start = off_e[e]
        end = off_e[e] + cnt_e[e]
        nonempty = cnt_e[e] > 0
        fb = start // GBLK
        lb = jnp.where(nonempty, (end - 1) // GBLK, 0)
        nb = jnp.where(nonempty, lb - fb + 1, 0)
        active = (lane_s >= cum) & (lane_s < cum + nb)
        rb_here = fb + (lane_s - cum)
        rb_row = rb_row + jnp.where(active, rb_here, 0)
        e_row = e_row + jnp.where(active, e, 0)
        st_row = st_row + jnp.where(active, jnp.maximum(start, rb_here * GBLK), 0)
        en_row = en_row + jnp.where(active, jnp.minimum(end, (rb_here + 1) * GBLK), 0)
        any_row = any_row + active.astype(jnp.int32)
        cum = cum + nb
    rb_row = jnp.where(any_row > 0, rb_row, NGB - 1)
    prev = jnp.concatenate(
        [jnp.full((1, 1), -1, jnp.int32), rb_row[:, :NSTEP - 1]], axis=1)
    first_row = (rb_row != prev).astype(jnp.int32)
    meta_ref[...] = jnp.concatenate(
        [rb_row, e_row, first_row, st_row, en_row], axis=0)


def _positions(i1, i2):
    return pl.pallas_call(
        _pos_body,
        grid=(1,),
        in_specs=[
            pl.BlockSpec((N_TOK, 1), lambda i: (0, 0)),
            pl.BlockSpec((N_TOK, 1), lambda i: (0, 0)),
        ],
        out_specs=[
            pl.BlockSpec((N_TOK, 1), lambda i: (0, 0)),
            pl.BlockSpec((N_TOK, 1), lambda i: (0, 0)),
            pl.BlockSpec((5, NSTEP), lambda i: (0, 0)),
        ],
        out_shape=[
            jax.ShapeDtypeStruct((N_TOK, 1), jnp.int32),
            jax.ShapeDtypeStruct((N_TOK, 1), jnp.int32),
            jax.ShapeDtypeStruct((5, NSTEP), jnp.int32),
        ],
    )(i1, i2)


_SC_CH = 64


def _sc_mesh():
    return plsc.VectorSubcoreMesh(core_axis_name="c", subcore_axis_name="s",
                                  num_cores=_NC, num_subcores=_NS)


def _sc_dispatch(x1, p1f, p2f):
    tok_per_w = N_TOK // _NW

    def body(x_hbm, p1_hbm, p2_hbm, xs_hbm, idx_v, rows_v, sem):
        wid = jax.lax.axis_index("s") * _NC + jax.lax.axis_index("c")
        base0 = wid * tok_per_w
        for c in range(tok_per_w // _SC_CH):
            base = base0 + c * _SC_CH
            pltpu.sync_copy(x_hbm.at[pl.ds(base, _SC_CH)], rows_v)
            pltpu.sync_copy(p1_hbm.at[pl.ds(base, _SC_CH)], idx_v)
            pltpu.async_copy(rows_v, xs_hbm.at[idx_v], sem).wait()
            pltpu.sync_copy(p2_hbm.at[pl.ds(base, _SC_CH)], idx_v)
            pltpu.async_copy(rows_v, xs_hbm.at[idx_v], sem).wait()

    f = pl.kernel(
        body,
        out_type=jax.ShapeDtypeStruct((N_SLOT, D_MODEL), jnp.float32),
        mesh=_sc_mesh(),
        scratch_types=[
            pltpu.VMEM((_SC_CH,), jnp.int32),
            pltpu.VMEM((_SC_CH, D_MODEL), jnp.float32),
            pltpu.SemaphoreType.DMA,
        ],
    )
    return f(x1, p1f, p2f)


def _sc_gather2(ys, p1f, p2f):
    tok_per_w = N_TOK // _NW

    def body(ys_hbm, p1_hbm, p2_hbm, o1_hbm, o2_hbm, idx_v, rows_v, sem):
        wid = jax.lax.axis_index("s") * _NC + jax.lax.axis_index("c")
        base0 = wid * tok_per_w
        for c in range(tok_per_w // _SC_CH):
            base = base0 + c * _SC_CH
            pltpu.sync_copy(p1_hbm.at[pl.ds(base, _SC_CH)], idx_v)
            pltpu.async_copy(ys_hbm.at[idx_v], rows_v, sem).wait()
            pltpu.sync_copy(rows_v, o1_hbm.at[pl.ds(base, _SC_CH)])
            pltpu.sync_copy(p2_hbm.at[pl.ds(base, _SC_CH)], idx_v)
            pltpu.async_copy(ys_hbm.at[idx_v], rows_v, sem).wait()
            pltpu.sync_copy(rows_v, o2_hbm.at[pl.ds(base, _SC_CH)])

    f = pl.kernel(
        body,
        out_type=[
            jax.ShapeDtypeStruct((N_TOK, D_MODEL), jnp.float32),
            jax.ShapeDtypeStruct((N_TOK, D_MODEL), jnp.float32),
        ],
        mesh=_sc_mesh(),
        scratch_types=[
            pltpu.VMEM((_SC_CH,), jnp.int32),
            pltpu.VMEM((_SC_CH, D_MODEL), jnp.float32),
            pltpu.SemaphoreType.DMA,
        ],
    )
    return f(ys, p1f, p2f)


def _gmm_body(meta_ref, xs_ref, we1_ref, be1_ref, we2_ref, be2_ref, ys_ref):
    s = pl.program_id(0)
    rb = meta_ref[0, s]
    first = meta_ref[2, s]
    start = meta_ref[3, s]
    end = meta_ref[4, s]
    x = xs_ref[...].astype(jnp.bfloat16)
    h = jnp.dot(x, we1_ref[0], preferred_element_type=jnp.float32)
    h = jnp.maximum(h + be1_ref[0], 0.0)
    y = jnp.dot(h.astype(jnp.bfloat16), we2_ref[0],
                preferred_element_type=jnp.float32)
    y = y + be2_ref[0]
    gi = rb * GBLK + jax.lax.broadcasted_iota(jnp.int32, (GBLK, 1), 0)
    rowmask = (gi >= start) & (gi < end)
    contrib = jnp.where(rowmask, y, 0.0)

    @pl.when(first == 1)
    def _init():
        ys_ref[...] = contrib

    @pl.when(first == 0)
    def _acc():
        ys_ref[...] = ys_ref[...] + contrib


def _gmm(meta, xs, we1b, we2b, p):
    grid_spec = pltpu.PrefetchScalarGridSpec(
        num_scalar_prefetch=1,
        grid=(NSTEP,),
        in_specs=[
            pl.BlockSpec((GBLK, D_MODEL), lambda s, m: (m[0, s], 0)),
            pl.BlockSpec((1, D_MODEL, D_FF), lambda s, m: (m[1, s], 0, 0)),
            pl.BlockSpec((1, 1, D_FF), lambda s, m: (m[1, s], 0, 0)),
            pl.BlockSpec((1, D_FF, D_MODEL), lambda s, m: (m[1, s], 0, 0)),
            pl.BlockSpec((1, 1, D_MODEL), lambda s, m: (m[1, s], 0, 0)),
        ],
        out_specs=pl.BlockSpec((GBLK, D_MODEL), lambda s, m: (m[0, s], 0)),
    )
    return pl.pallas_call(
        _gmm_body,
        grid_spec=grid_spec,
        out_shape=jax.ShapeDtypeStruct((N_SLOT, D_MODEL), jnp.float32),
    )(meta, xs, we1b, p['be1'].reshape(N_EXPERTS, 1, D_FF),
      we2b, p['be2'].reshape(N_EXPERTS, 1, D_MODEL))


def _cln2_body(o1_ref, o2_ref, w1_ref, w2_ref, xres_ref, g_ref, b_ref,
               mask_ref, out_ref):
    y = w1_ref[...] * o1_ref[...] + w2_ref[...] * o2_ref[...] + xres_ref[...]
    m = jnp.mean(y, axis=1, keepdims=True)
    v = jnp.mean((y - m) ** 2, axis=1, keepdims=True)
    x2 = (y - m) / jnp.sqrt(v + 1e-5) * g_ref[...] + b_ref[...]
    out_ref[...] = x2 * mask_ref[...]


def _combine_ln2(o1, o2, w1, w2, xres, p, mask):
    blk = 512
    return pl.pallas_call(
        _cln2_body,
        grid=(N_TOK // blk,),
        in_specs=[
            pl.BlockSpec((blk, D_MODEL), lambda i: (i, 0)),
            pl.BlockSpec((blk, D_MODEL), lambda i: (i, 0)),
            pl.BlockSpec((blk, 1), lambda i: (i, 0)),
            pl.BlockSpec((blk, 1), lambda i: (i, 0)),
            pl.BlockSpec((blk, D_MODEL), lambda i: (i, 0)),
            pl.BlockSpec((1, D_MODEL), lambda i: (0, 0)),
            pl.BlockSpec((1, D_MODEL), lambda i: (0, 0)),
            pl.BlockSpec((blk, 1), lambda i: (i, 0)),
        ],
        out_specs=pl.BlockSpec((blk, D_MODEL), lambda i: (i, 0)),
        out_shape=jax.ShapeDtypeStruct((N_TOK, D_MODEL), jnp.float32),
    )(o1, o2, w1, w2, xres, p['g2'].reshape(1, -1), p['b2'].reshape(1, -1),
      mask)


def kernel(frac, params, src):
    p = params
    frac2d = frac.reshape(N_TOK, 1)
    fracr = frac.reshape(N_TOK // ATT_ROWS, 1, ATT_ROWS)
    src2d = src.reshape(N_TOK, 1).astype(jnp.int32)
    wqkv = jnp.concatenate([p['Wq'], p['Wk'], p['Wv']], axis=1)
    bqkv = jnp.concatenate([p['bq'], p['bk'], p['bv']]).reshape(1, -1)
    ones = jnp.ones((N_TOK, 1), jnp.float32)
    finalmask = (frac2d != 0.0).astype(jnp.float32)
    we1b = p['We1'].astype(jnp.bfloat16)
    we2b = p['We2'].astype(jnp.bfloat16)
    wqkv = wqkv.astype(jnp.bfloat16)
    wob = p['Wo'].astype(jnp.bfloat16)
    wgb = p['Wg'].astype(jnp.bfloat16)

    ctx, x = _attn_first(src2d, frac2d, fracr, p, wqkv, bqkv)
    for layer in range(3):
        x1, i1, i2, w1, w2 = _post(ctx, x, p, wob, p['bo'].reshape(1, -1),
                                   wgb)
        p1, p2, meta = _positions(i1, i2)
        p1f = p1.reshape(N_TOK)
        p2f = p2.reshape(N_TOK)
        xs = _sc_dispatch(x1, p1f, p2f)
        ys = _gmm(meta, xs, we1b, we2b, p)
        o1, o2 = _sc_gather2(ys, p1f, p2f)
        if layer < 2:
            ctx, x = _attn_mid(o1, o2, w1, w2, x1, frac2d, fracr, p,
                               wqkv, bqkv)
        else:
            x = _combine_ln2(o1, o2, w1, w2, x1, p, finalmask)
    return x.reshape(B, T, D_MODEL)


# speedup vs baseline: 1.8727x; 1.0207x over previous
import functools

import jax
import jax.numpy as jnp
import numpy as np
from jax.experimental import pallas as pl
from jax.experimental.pallas import tpu as pltpu
from jax.experimental.pallas import tpu_sc as plsc

D_MODEL = 1024
N_HEADS = 16
HEAD_DIM = 64
N_EXPERTS = 8
D_FF = 2048
N_BASIS = 10
DEGREE = 3
VOCAB = 120
FEAT = 200
B = 512
T = 8
N_TOK = B * T
N_SLOT = 2 * N_TOK
GBLK = 256
NGB = N_SLOT // GBLK
NSTEP = NGB + N_EXPERTS

_NC, _NS = 2, 16
_NW = _NC * _NS

_base = np.linspace(0.0, 1.0, N_BASIS + DEGREE + 1 - 2 * DEGREE)
_KNOTS = np.concatenate(
    [np.repeat(_base[:1], DEGREE), _base, np.repeat(_base[-1:], DEGREE)]
).astype(np.float64)


def _bspline_basis(f):
    nk = _KNOTS.shape[0]
    Bp = [
        jnp.where((f >= float(_KNOTS[i])) & (f < float(_KNOTS[i + 1])), 1.0, 0.0)
        for i in range(nk - 1)
    ]
    for d in range(1, DEGREE + 1):
        Bc = []
        for i in range(nk - d - 1):
            den1 = float(_KNOTS[i + d] - _KNOTS[i])
            den2 = float(_KNOTS[i + d + 1] - _KNOTS[i + 1])
            t = jnp.zeros_like(f)
            if den1 != 0.0:
                t = t + (f - float(_KNOTS[i])) / den1 * Bp[i]
            if den2 != 0.0:
                t = t + (float(_KNOTS[i + d + 1]) - f) / den2 * Bp[i + 1]
            Bc.append(t)
        Bp = Bc
    return jnp.concatenate(Bp, axis=1)


ATT_ROWS = 256


def _embed_rows(src, frac, cbfv_ref, we_ref, be_ref, wpe_ref, bpe_ref,
                wple_ref, bple_ref, sc_ref):
    rows = src.shape[0]
    oh = (src == jax.lax.broadcasted_iota(jnp.int32, (rows, VOCAB), 1)).astype(
        jnp.float32)
    feats = jnp.dot(oh, cbfv_ref[...], preferred_element_type=jnp.float32)
    x = jnp.dot(feats, we_ref[...], preferred_element_type=jnp.float32)
    x = x + be_ref[...]
    emb_scaler = sc_ref[0, 0]
    pos_scaler = sc_ref[0, 1]
    pos_scaler_log = sc_ref[0, 2]
    x = x * jnp.exp2(emb_scaler)
    pe_scaler = jnp.exp2((1.0 - pos_scaler) ** 2)
    ple_scaler = jnp.exp2((1.0 - pos_scaler_log) ** 2)

    f = jnp.clip(frac, 1e-9, 1.0)
    basis = _bspline_basis(f)
    pe = (jnp.dot(basis, wpe_ref[...], preferred_element_type=jnp.float32)
          + bpe_ref[...]) * pe_scaler
    f2 = jnp.clip(0.0025 * jnp.log2(f) ** 2, 0.0, 1.0)
    basis2 = _bspline_basis(f2)
    ple = (jnp.dot(basis2, wple_ref[...], preferred_element_type=jnp.float32)
           + bple_ref[...]) * ple_scaler
    return x + jnp.concatenate([pe, ple], axis=1)


def _attn_core(x, frac_ref, fracr_ref, wqkv_ref, bqkv_ref, alpha_ref, out_ref):
    qkv = jnp.dot(x.astype(jnp.bfloat16), wqkv_ref[...],
                  preferred_element_type=jnp.float32)
    qkv = qkv + bqkv_ref[...]
    qkvb = qkv.astype(jnp.bfloat16)
    fcol = frac_ref[...]
    frow = fracr_ref[0]
    alpha = alpha_ref[0, 0]
    R = ATT_ROWS
    diff = fcol - frow
    bias = alpha * (jnp.log1p(jnp.abs(diff)) * jnp.sign(diff))
    ii = jax.lax.broadcasted_iota(jnp.int32, (R, R), 0)
    jj = jax.lax.broadcasted_iota(jnp.int32, (R, R), 1)
    same = (ii // T) == (jj // T)
    keyok = frow != 0.0
    valid = same & keyok
    scale = HEAD_DIM ** -0.5
    for h in range(N_HEADS):
        q = qkvb[:, h * HEAD_DIM:(h + 1) * HEAD_DIM]
        k = qkvb[:, D_MODEL + h * HEAD_DIM:D_MODEL + (h + 1) * HEAD_DIM]
        v = qkvb[:, 2 * D_MODEL + h * HEAD_DIM:2 * D_MODEL + (h + 1) * HEAD_DIM]
        s = jax.lax.dot_general(q, k, (((1,), (1,)), ((), ())),
                                preferred_element_type=jnp.float32) * scale
        s = jnp.where(valid, s + bias, -1e30)
        m = jnp.max(s, axis=1, keepdims=True)
        e = jnp.exp(s - m)
        pr = e * (1.0 / jnp.sum(e, axis=1, keepdims=True))
        ctx = jnp.dot(pr.astype(jnp.bfloat16), v,
                      preferred_element_type=jnp.float32)
        out_ref[:, h * HEAD_DIM:(h + 1) * HEAD_DIM] = ctx


def _attn_first_body(src_ref, frac_ref, fracr_ref, cbfv_ref, we_ref, be_ref,
                     wpe_ref, bpe_ref, wple_ref, bple_ref, sc_ref,
                     wqkv_ref, bqkv_ref, alpha_ref, ctx_ref, x0_ref):
    x0 = _embed_rows(src_ref[...], frac_ref[...], cbfv_ref, we_ref, be_ref,
                     wpe_ref, bpe_ref, wple_ref, bple_ref, sc_ref)
    x0_ref[...] = x0
    _attn_core(x0, frac_ref, fracr_ref, wqkv_ref, bqkv_ref, alpha_ref, ctx_ref)


def _attn_first(src2d, frac2d, fracr, p, wqkv, bqkv):
    grid = (N_TOK // ATT_ROWS,)
    half = D_MODEL // 2
    scalars = jnp.stack([p['emb_scaler'], p['pos_scaler'],
                         p['pos_scaler_log']]).reshape(1, 3)
    full = lambda *shape: pl.BlockSpec(shape, lambda i: (0,) * len(shape))
    return pl.pallas_call(
        _attn_first_body,
        grid=grid,
        in_specs=[
            pl.BlockSpec((ATT_ROWS, 1), lambda i: (i, 0)),
            pl.BlockSpec((ATT_ROWS, 1), lambda i: (i, 0)),
            pl.BlockSpec((1, 1, ATT_ROWS), lambda i: (i, 0, 0)),
            full(VOCAB, FEAT),
            full(FEAT, D_MODEL),
            full(1, D_MODEL),
            full(N_BASIS, half),
            full(1, half),
            full(N_BASIS, half),
            full(1, half),
            full(1, 3),
            full(D_MODEL, 3 * D_MODEL),
            full(1, 3 * D_MODEL),
            full(1, 1),
        ],
        out_specs=[
            pl.BlockSpec((ATT_ROWS, D_MODEL), lambda i: (i, 0)),
            pl.BlockSpec((ATT_ROWS, D_MODEL), lambda i: (i, 0)),
        ],
        out_shape=[
            jax.ShapeDtypeStruct((N_TOK, D_MODEL), jnp.float32),
            jax.ShapeDtypeStruct((N_TOK, D_MODEL), jnp.float32),
        ],
    )(src2d, frac2d, fracr, p['cbfv'], p['We'], p['be'].reshape(1, -1),
      p['W_pe'], p['b_pe'].reshape(1, -1), p['W_ple'],
      p['b_ple'].reshape(1, -1), scalars, wqkv, bqkv,
      p['alpha'].reshape(1, 1))


def _attn_mid_body(o1_ref, o2_ref, w1_ref, w2_ref, xres_ref, g2_ref, b2_ref,
                   frac_ref, fracr_ref, wqkv_ref, bqkv_ref, alpha_ref,
                   ctx_ref, x_ref):
    y = (w1_ref[...] * o1_ref[...] + w2_ref[...] * o2_ref[...]
         + xres_ref[...])
    m = jnp.mean(y, axis=1, keepdims=True)
    v = jnp.mean((y - m) ** 2, axis=1, keepdims=True)
    x = (y - m) / jnp.sqrt(v + 1e-5) * g2_ref[...] + b2_ref[...]
    x_ref[...] = x
    _attn_core(x, frac_ref, fracr_ref, wqkv_ref, bqkv_ref, alpha_ref, ctx_ref)


def _attn_mid(o1, o2, w1, w2, xres, frac2d, fracr, p, wqkv, bqkv):
    grid = (N_TOK // ATT_ROWS,)
    full = lambda *shape: pl.BlockSpec(shape, lambda i: (0,) * len(shape))
    return pl.pallas_call(
        _attn_mid_body,
        grid=grid,
        in_specs=[
            pl.BlockSpec((ATT_ROWS, D_MODEL), lambda i: (i, 0)),
            pl.BlockSpec((ATT_ROWS, D_MODEL), lambda i: (i, 0)),
            pl.BlockSpec((ATT_ROWS, 1), lambda i: (i, 0)),
            pl.BlockSpec((ATT_ROWS, 1), lambda i: (i, 0)),
            pl.BlockSpec((ATT_ROWS, D_MODEL), lambda i: (i, 0)),
            full(1, D_MODEL),
            full(1, D_MODEL),
            pl.BlockSpec((ATT_ROWS, 1), lambda i: (i, 0)),
            pl.BlockSpec((1, 1, ATT_ROWS), lambda i: (i, 0, 0)),
            full(D_MODEL, 3 * D_MODEL),
            full(1, 3 * D_MODEL),
            full(1, 1),
        ],
        out_specs=[
            pl.BlockSpec((ATT_ROWS, D_MODEL), lambda i: (i, 0)),
            pl.BlockSpec((ATT_ROWS, D_MODEL), lambda i: (i, 0)),
        ],
        out_shape=[
            jax.ShapeDtypeStruct((N_TOK, D_MODEL), jnp.float32),
            jax.ShapeDtypeStruct((N_TOK, D_MODEL), jnp.float32),
        ],
    )(o1, o2, w1, w2, xres, p['g2'].reshape(1, -1), p['b2'].reshape(1, -1),
      frac2d, fracr, wqkv, bqkv, p['alpha'].reshape(1, 1))


def _post_body(ctx_ref, xin_ref, wo_ref, bo_ref, g1_ref, b1_ref, wg_ref,
               bg_ref, x1_ref, xp_ref, i1_ref, i2_ref, w1_ref, w2_ref):
    y = jnp.dot(ctx_ref[...].astype(jnp.bfloat16), wo_ref[...],
                preferred_element_type=jnp.float32)
    y = y + bo_ref[...] + xin_ref[...]
    m = jnp.mean(y, axis=1, keepdims=True)
    v = jnp.mean((y - m) ** 2, axis=1, keepdims=True)
    x1 = (y - m) / jnp.sqrt(v + 1e-5) * g1_ref[...] + b1_ref[...]
    x1_ref[...] = x1
    half = D_MODEL // 2
    hi = jax.lax.bitcast_convert_type(
        x1[:, :half].astype(jnp.bfloat16).astype(jnp.float32), jnp.uint32)
    lo = jax.lax.bitcast_convert_type(
        x1[:, half:].astype(jnp.bfloat16).astype(jnp.float32), jnp.uint32)
    xp_ref[...] = jax.lax.bitcast_convert_type(hi | (lo >> 16), jnp.int32)
    logits = jnp.dot(x1.astype(jnp.bfloat16), wg_ref[...],
                     preferred_element_type=jnp.float32)
    logits = logits + bg_ref[...]
    lm = jnp.max(logits, axis=1, keepdims=True)
    le = jnp.exp(logits - lm)
    probs = le / jnp.sum(le, axis=1, keepdims=True)
    rows = probs.shape[0]
    lane = jax.lax.broadcasted_iota(jnp.int32, (rows, N_EXPERTS), 1)
    w1 = jnp.max(probs, axis=1, keepdims=True)
    i1 = jnp.min(jnp.where(probs == w1, lane, N_EXPERTS), axis=1, keepdims=True)
    probs2 = jnp.where(lane == i1, -1.0, probs)
    w2 = jnp.max(probs2, axis=1, keepdims=True)
    i2 = jnp.min(jnp.where(probs2 == w2, lane, N_EXPERTS), axis=1, keepdims=True)
    i1_ref[...] = i1
    i2_ref[...] = i2
    w1_ref[...] = w1
    w2_ref[...] = w2


def _post(ctx, xin, p, wo, bo, wg):
    blk = 512
    grid = (N_TOK // blk,)
    return pl.pallas_call(
        _post_body,
        grid=grid,
        in_specs=[
            pl.BlockSpec((blk, D_MODEL), lambda i: (i, 0)),
            pl.BlockSpec((blk, D_MODEL), lambda i: (i, 0)),
            pl.BlockSpec((D_MODEL, D_MODEL), lambda i: (0, 0)),
            pl.BlockSpec((1, D_MODEL), lambda i: (0, 0)),
            pl.BlockSpec((1, D_MODEL), lambda i: (0, 0)),
            pl.BlockSpec((1, D_MODEL), lambda i: (0, 0)),
            pl.BlockSpec((D_MODEL, N_EXPERTS), lambda i: (0, 0)),
            pl.BlockSpec((1, N_EXPERTS), lambda i: (0, 0)),
        ],
        out_specs=[
            pl.BlockSpec((blk, D_MODEL), lambda i: (i, 0)),
            pl.BlockSpec((blk, D_MODEL // 2), lambda i: (i, 0)),
            pl.BlockSpec((blk, 1), lambda i: (i, 0)),
            pl.BlockSpec((blk, 1), lambda i: (i, 0)),
            pl.BlockSpec((blk, 1), lambda i: (i, 0)),
            pl.BlockSpec((blk, 1), lambda i: (i, 0)),
        ],
        out_shape=[
            jax.ShapeDtypeStruct((N_TOK, D_MODEL), jnp.float32),
            jax.ShapeDtypeStruct((N_TOK, D_MODEL // 2), jnp.int32),
            jax.ShapeDtypeStruct((N_TOK, 1), jnp.int32),
            jax.ShapeDtypeStruct((N_TOK, 1), jnp.int32),
            jax.ShapeDtypeStruct((N_TOK, 1), jnp.float32),
            jax.ShapeDtypeStruct((N_TOK, 1), jnp.float32),
        ],
    )(ctx, xin, wo, bo, p['g1'].reshape(1, -1), p['b1'].reshape(1, -1),
      wg, p['bg'].reshape(1, -1))


def _cumsum_rows(a):
    n, w = a.shape
    sh = 1
    while sh < n:
        a = a + jnp.concatenate(
            [jnp.zeros((sh, w), a.dtype), a[:-sh]], axis=0)
        sh *= 2
    return a


def _pos_body(i1_ref, i2_ref, p1_ref, p2_ref, meta_ref):
    lane = jax.lax.broadcasted_iota(jnp.int32, (N_TOK, N_EXPERTS), 1)
    h1 = (i1_ref[...] == lane).astype(jnp.int32)
    h2 = (i2_ref[...] == lane).astype(jnp.int32)
    c1 = _cumsum_rows(h1)
    c2 = _cumsum_rows(h2)
    cnt1 = c1[N_TOK - 1:N_TOK, :]
    cnt2 = c2[N_TOK - 1:N_TOK, :]
    counts = cnt1 + cnt2
    lane8 = jax.lax.broadcasted_iota(jnp.int32, (1, N_EXPERTS), 1)

    offs_row = jnp.zeros((1, N_EXPERTS), jnp.int32)
    off = jnp.int32(0)
    off_e = []
    cnt_e = []
    cnt1_e = []
    for e in range(N_EXPERTS):
        ce = jnp.sum(jnp.where(lane8 == e, counts, 0))
        c1e = jnp.sum(jnp.where(lane8 == e, cnt1, 0))
        off_e.append(off)
        cnt_e.append(ce)
        cnt1_e.append(c1e)
        offs_row = offs_row + jnp.where(lane8 == e, off, 0)
        off = off + ce

    cnt1_row = cnt1
    p1_ref[...] = jnp.sum(h1 * (offs_row + c1 - h1), axis=1, keepdims=True)
    p2_ref[...] = jnp.sum(h2 * (offs_row + cnt1_row + c2 - h2), axis=1,
                          keepdims=True)

    lane_s = jax.lax.broadcasted_iota(jnp.int32, (1, NSTEP), 1)
    rb_row = jnp.zeros((1, NSTEP), jnp.int32)
    e_row = jnp.zeros((1, NSTEP), jnp.int32)
    st_row = jnp.zeros((1, NSTEP), jnp.int32)
    en_row = jnp.zeros((1, NSTEP), jnp.int32)
    any_row = jnp.zeros((1, NSTEP), jnp.int32)
    cum = jnp.int32(0)
    for e in range(N_EXPERTS):
        start = off_e[e]
        end = off_e[e] + cnt_e[e]
        nonempty = cnt_e[e] > 0
        fb = start // GBLK
        lb = jnp.where(nonempty, (end - 1) // GBLK, 0)
        nb = jnp.where(nonempty, lb - fb + 1, 0)
        active = (lane_s >= cum) & (lane_s < cum + nb)
        rb_here = fb + (lane_s - cum)
        rb_row = rb_row + jnp.where(active, rb_here, 0)
        e_row = e_row + jnp.where(active, e, 0)
        st_row = st_row + jnp.where(active, jnp.maximum(start, rb_here * GBLK), 0)
        en_row = en_row + jnp.where(active, jnp.minimum(end, (rb_here + 1) * GBLK), 0)
        any_row = any_row + active.astype(jnp.int32)
        cum = cum + nb
    rb_row = jnp.where(any_row > 0, rb_row, NGB - 1)
    prev = jnp.concatenate(
        [jnp.full((1, 1), -1, jnp.int32), rb_row[:, :NSTEP - 1]], axis=1)
    first_row = (rb_row != prev).astype(jnp.int32)
    meta_ref[...] = jnp.concatenate(
        [rb_row, e_row, first_row, st_row, en_row], axis=0)


def _positions(i1, i2):
    return pl.pallas_call(
        _pos_body,
        grid=(1,),
        in_specs=[
            pl.BlockSpec((N_TOK, 1), lambda i: (0, 0)),
            pl.BlockSpec((N_TOK, 1), lambda i: (0, 0)),
        ],
        out_specs=[
            pl.BlockSpec((N_TOK, 1), lambda i: (0, 0)),
            pl.BlockSpec((N_TOK, 1), lambda i: (0, 0)),
            pl.BlockSpec((5, NSTEP), lambda i: (0, 0)),
        ],
        out_shape=[
            jax.ShapeDtypeStruct((N_TOK, 1), jnp.int32),
            jax.ShapeDtypeStruct((N_TOK, 1), jnp.int32),
            jax.ShapeDtypeStruct((5, NSTEP), jnp.int32),
        ],
    )(i1, i2)


_SC_CH = 64


def _sc_mesh():
    return plsc.VectorSubcoreMesh(core_axis_name="c", subcore_axis_name="s",
                                  num_cores=_NC, num_subcores=_NS)


def _sc_dispatch(xp, p1f, p2f):
    tok_per_w = N_TOK // _NW
    width = D_MODEL // 2

    def body(x_hbm, p1_hbm, p2_hbm, xs_hbm, idx_v, rows_v, sem):
        wid = jax.lax.axis_index("s") * _NC + jax.lax.axis_index("c")
        base0 = wid * tok_per_w
        for c in range(tok_per_w // _SC_CH):
            base = base0 + c * _SC_CH
            pltpu.sync_copy(x_hbm.at[pl.ds(base, _SC_CH)], rows_v)
            pltpu.sync_copy(p1_hbm.at[pl.ds(base, _SC_CH)], idx_v)
            pltpu.async_copy(rows_v, xs_hbm.at[idx_v], sem).wait()
            pltpu.sync_copy(p2_hbm.at[pl.ds(base, _SC_CH)], idx_v)
            pltpu.async_copy(rows_v, xs_hbm.at[idx_v], sem).wait()

    f = pl.kernel(
        body,
        out_type=jax.ShapeDtypeStruct((N_SLOT, width), jnp.int32),
        mesh=_sc_mesh(),
        scratch_types=[
            pltpu.VMEM((_SC_CH,), jnp.int32),
            pltpu.VMEM((_SC_CH, width), jnp.int32),
            pltpu.SemaphoreType.DMA,
        ],
    )
    return f(xp, p1f, p2f)


def _sc_gather2(ys, p1f, p2f):
    tok_per_w = N_TOK // _NW

    def body(ys_hbm, p1_hbm, p2_hbm, o1_hbm, o2_hbm, idx_v, rows_v, sem):
        wid = jax.lax.axis_index("s") * _NC + jax.lax.axis_index("c")
        base0 = wid * tok_per_w
        for c in range(tok_per_w // _SC_CH):
            base = base0 + c * _SC_CH
            pltpu.sync_copy(p1_hbm.at[pl.ds(base, _SC_CH)], idx_v)
            pltpu.async_copy(ys_hbm.at[idx_v], rows_v, sem).wait()
            pltpu.sync_copy(rows_v, o1_hbm.at[pl.ds(base, _SC_CH)])
            pltpu.sync_copy(p2_hbm.at[pl.ds(base, _SC_CH)], idx_v)
            pltpu.async_copy(ys_hbm.at[idx_v], rows_v, sem).wait()
            pltpu.sync_copy(rows_v, o2_hbm.at[pl.ds(base, _SC_CH)])

    f = pl.kernel(
        body,
        out_type=[
            jax.ShapeDtypeStruct((N_TOK, D_MODEL), jnp.float32),
            jax.ShapeDtypeStruct((N_TOK, D_MODEL), jnp.float32),
        ],
        mesh=_sc_mesh(),
        scratch_types=[
            pltpu.VMEM((_SC_CH,), jnp.int32),
            pltpu.VMEM((_SC_CH, D_MODEL), jnp.float32),
            pltpu.SemaphoreType.DMA,
        ],
    )
    return f(ys, p1f, p2f)


def _gmm_body(meta_ref, xs_ref, we1_ref, be1_ref, we2_ref, be2_ref, ys_ref):
    s = pl.program_id(0)
    rb = meta_ref[0, s]
    first = meta_ref[2, s]
    start = meta_ref[3, s]
    end = meta_ref[4, s]
    packed = jax.lax.bitcast_convert_type(xs_ref[...], jnp.uint32)
    xhi = jax.lax.bitcast_convert_type(
        packed & jnp.uint32(0xFFFF0000), jnp.float32).astype(jnp.bfloat16)
    xlo = jax.lax.bitcast_convert_type(
        packed << 16, jnp.float32).astype(jnp.bfloat16)
    half = D_MODEL // 2
    h = (jnp.dot(xhi, we1_ref[0, :half], preferred_element_type=jnp.float32)
         + jnp.dot(xlo, we1_ref[0, half:],
                   preferred_element_type=jnp.float32))
    h = jnp.maximum(h + be1_ref[0], 0.0)
    y = jnp.dot(h.astype(jnp.bfloat16), we2_ref[0],
                preferred_element_type=jnp.float32)
    y = y + be2_ref[0]
    gi = rb * GBLK + jax.lax.broadcasted_iota(jnp.int32, (GBLK, 1), 0)
    rowmask = (gi >= start) & (gi < end)
    contrib = jnp.where(rowmask, y, 0.0)

    @pl.when(first == 1)
    def _init():
        ys_ref[...] = contrib

    @pl.when(first == 0)
    def _acc():
        ys_ref[...] = ys_ref[...] + contrib


def _gmm(meta, xs, we1b, we2b, p):
    grid_spec = pltpu.PrefetchScalarGridSpec(
        num_scalar_prefetch=1,
        grid=(NSTEP,),
        in_specs=[
            pl.BlockSpec((GBLK, D_MODEL // 2), lambda s, m: (m[0, s], 0)),
            pl.BlockSpec((1, D_MODEL, D_FF), lambda s, m: (m[1, s], 0, 0)),
            pl.BlockSpec((1, 1, D_FF), lambda s, m: (m[1, s], 0, 0)),
            pl.BlockSpec((1, D_FF, D_MODEL), lambda s, m: (m[1, s], 0, 0)),
            pl.BlockSpec((1, 1, D_MODEL), lambda s, m: (m[1, s], 0, 0)),
        ],
        out_specs=pl.BlockSpec((GBLK, D_MODEL), lambda s, m: (m[0, s], 0)),
    )
    return pl.pallas_call(
        _gmm_body,
        grid_spec=grid_spec,
        out_shape=jax.ShapeDtypeStruct((N_SLOT, D_MODEL), jnp.float32),
    )(meta, xs, we1b, p['be1'].reshape(N_EXPERTS, 1, D_FF),
      we2b, p['be2'].reshape(N_EXPERTS, 1, D_MODEL))


def _cln2_body(o1_ref, o2_ref, w1_ref, w2_ref, xres_ref, g_ref, b_ref,
               mask_ref, out_ref):
    y = w1_ref[...] * o1_ref[...] + w2_ref[...] * o2_ref[...] + xres_ref[...]
    m = jnp.mean(y, axis=1, keepdims=True)
    v = jnp.mean((y - m) ** 2, axis=1, keepdims=True)
    x2 = (y - m) / jnp.sqrt(v + 1e-5) * g_ref[...] + b_ref[...]
    out_ref[...] = x2 * mask_ref[...]


def _combine_ln2(o1, o2, w1, w2, xres, p, mask):
    blk = 512
    return pl.pallas_call(
        _cln2_body,
        grid=(N_TOK // blk,),
        in_specs=[
            pl.BlockSpec((blk, D_MODEL), lambda i: (i, 0)),
            pl.BlockSpec((blk, D_MODEL), lambda i: (i, 0)),
            pl.BlockSpec((blk, 1), lambda i: (i, 0)),
            pl.BlockSpec((blk, 1), lambda i: (i, 0)),
            pl.BlockSpec((blk, D_MODEL), lambda i: (i, 0)),
            pl.BlockSpec((1, D_MODEL), lambda i: (0, 0)),
            pl.BlockSpec((1, D_MODEL), lambda i: (0, 0)),
            pl.BlockSpec((blk, 1), lambda i: (i, 0)),
        ],
        out_specs=pl.BlockSpec((blk, D_MODEL), lambda i: (i, 0)),
        out_shape=jax.ShapeDtypeStruct((N_TOK, D_MODEL), jnp.float32),
    )(o1, o2, w1, w2, xres, p['g2'].reshape(1, -1), p['b2'].reshape(1, -1),
      mask)


def kernel(frac, params, src):
    p = params
    frac2d = frac.reshape(N_TOK, 1)
    fracr = frac.reshape(N_TOK // ATT_ROWS, 1, ATT_ROWS)
    src2d = src.reshape(N_TOK, 1).astype(jnp.int32)
    wqkv = jnp.concatenate([p['Wq'], p['Wk'], p['Wv']], axis=1)
    bqkv = jnp.concatenate([p['bq'], p['bk'], p['bv']]).reshape(1, -1)
    ones = jnp.ones((N_TOK, 1), jnp.float32)
    finalmask = (frac2d != 0.0).astype(jnp.float32)
    we1b = p['We1'].astype(jnp.bfloat16)
    we2b = p['We2'].astype(jnp.bfloat16)
    wqkv = wqkv.astype(jnp.bfloat16)
    wob = p['Wo'].astype(jnp.bfloat16)
    wgb = p['Wg'].astype(jnp.bfloat16)

    ctx, x = _attn_first(src2d, frac2d, fracr, p, wqkv, bqkv)
    for layer in range(3):
        x1, xp, i1, i2, w1, w2 = _post(ctx, x, p, wob,
                                       p['bo'].reshape(1, -1), wgb)
        p1, p2, meta = _positions(i1, i2)
        p1f = p1.reshape(N_TOK)
        p2f = p2.reshape(N_TOK)
        xs = _sc_dispatch(xp, p1f, p2f)
        ys = _gmm(meta, xs, we1b, we2b, p)
        o1, o2 = _sc_gather2(ys, p1f, p2f)
        if layer < 2:
            ctx, x = _attn_mid(o1, o2, w1, w2, x1, frac2d, fracr, p,
                               wqkv, bqkv)
        else:
            x = _combine_ln2(o1, o2, w1, w2, x1, p, finalmask)
    return x.reshape(B, T, D_MODEL)


# speedup vs baseline: 1.8734x; 1.0004x over previous
import functools

import jax
import jax.numpy as jnp
import numpy as np
from jax.experimental import pallas as pl
from jax.experimental.pallas import tpu as pltpu
from jax.experimental.pallas import tpu_sc as plsc

D_MODEL = 1024
N_HEADS = 16
HEAD_DIM = 64
N_EXPERTS = 8
D_FF = 2048
N_BASIS = 10
DEGREE = 3
VOCAB = 120
FEAT = 200
B = 512
T = 8
N_TOK = B * T
N_SLOT = 2 * N_TOK
GBLK = 256
NGB = N_SLOT // GBLK
NSTEP = NGB + N_EXPERTS

_NC, _NS = 2, 16
_NW = _NC * _NS

_base = np.linspace(0.0, 1.0, N_BASIS + DEGREE + 1 - 2 * DEGREE)
_KNOTS = np.concatenate(
    [np.repeat(_base[:1], DEGREE), _base, np.repeat(_base[-1:], DEGREE)]
).astype(np.float64)


def _bspline_basis(f):
    nk = _KNOTS.shape[0]
    Bp = [
        jnp.where((f >= float(_KNOTS[i])) & (f < float(_KNOTS[i + 1])), 1.0, 0.0)
        for i in range(nk - 1)
    ]
    for d in range(1, DEGREE + 1):
        Bc = []
        for i in range(nk - d - 1):
            den1 = float(_KNOTS[i + d] - _KNOTS[i])
            den2 = float(_KNOTS[i + d + 1] - _KNOTS[i + 1])
            t = jnp.zeros_like(f)
            if den1 != 0.0:
                t = t + (f - float(_KNOTS[i])) / den1 * Bp[i]
            if den2 != 0.0:
                t = t + (float(_KNOTS[i + d + 1]) - f) / den2 * Bp[i + 1]
            Bc.append(t)
        Bp = Bc
    return jnp.concatenate(Bp, axis=1)


ATT_ROWS = 256


def _embed_rows(src, frac, cbfv_ref, we_ref, be_ref, wpe_ref, bpe_ref,
                wple_ref, bple_ref, sc_ref):
    rows = src.shape[0]
    oh = (src == jax.lax.broadcasted_iota(jnp.int32, (rows, VOCAB), 1)).astype(
        jnp.float32)
    feats = jnp.dot(oh, cbfv_ref[...], preferred_element_type=jnp.float32)
    x = jnp.dot(feats, we_ref[...], preferred_element_type=jnp.float32)
    x = x + be_ref[...]
    emb_scaler = sc_ref[0, 0]
    pos_scaler = sc_ref[0, 1]
    pos_scaler_log = sc_ref[0, 2]
    x = x * jnp.exp2(emb_scaler)
    pe_scaler = jnp.exp2((1.0 - pos_scaler) ** 2)
    ple_scaler = jnp.exp2((1.0 - pos_scaler_log) ** 2)

    f = jnp.clip(frac, 1e-9, 1.0)
    basis = _bspline_basis(f)
    pe = (jnp.dot(basis, wpe_ref[...], preferred_element_type=jnp.float32)
          + bpe_ref[...]) * pe_scaler
    f2 = jnp.clip(0.0025 * jnp.log2(f) ** 2, 0.0, 1.0)
    basis2 = _bspline_basis(f2)
    ple = (jnp.dot(basis2, wple_ref[...], preferred_element_type=jnp.float32)
           + bple_ref[...]) * ple_scaler
    return x + jnp.concatenate([pe, ple], axis=1)


def _attn_core(x, frac_ref, fracr_ref, wqkv_ref, bqkv_ref, alpha_ref, out_ref):
    qkv = jnp.dot(x.astype(jnp.bfloat16), wqkv_ref[...],
                  preferred_element_type=jnp.float32)
    qkv = qkv + bqkv_ref[...]
    qkvb = qkv.astype(jnp.bfloat16)
    fcol = frac_ref[...]
    frow = fracr_ref[0]
    alpha = alpha_ref[0, 0]
    R = ATT_ROWS
    diff = fcol - frow
    bias = alpha * (jnp.log1p(jnp.abs(diff)) * jnp.sign(diff))
    ii = jax.lax.broadcasted_iota(jnp.int32, (R, R), 0)
    jj = jax.lax.broadcasted_iota(jnp.int32, (R, R), 1)
    same = (ii // T) == (jj // T)
    keyok = frow != 0.0
    valid = same & keyok
    scale = HEAD_DIM ** -0.5
    for h in range(N_HEADS):
        q = qkvb[:, h * HEAD_DIM:(h + 1) * HEAD_DIM]
        k = qkvb[:, D_MODEL + h * HEAD_DIM:D_MODEL + (h + 1) * HEAD_DIM]
        v = qkvb[:, 2 * D_MODEL + h * HEAD_DIM:2 * D_MODEL + (h + 1) * HEAD_DIM]
        s = jax.lax.dot_general(q, k, (((1,), (1,)), ((), ())),
                                preferred_element_type=jnp.float32) * scale
        s = jnp.where(valid, s + bias, -1e30)
        m = jnp.max(s, axis=1, keepdims=True)
        e = jnp.exp(s - m)
        pr = e * (1.0 / jnp.sum(e, axis=1, keepdims=True))
        ctx = jnp.dot(pr.astype(jnp.bfloat16), v,
                      preferred_element_type=jnp.float32)
        out_ref[:, h * HEAD_DIM:(h + 1) * HEAD_DIM] = ctx


def _attn_first_body(src_ref, frac_ref, fracr_ref, cbfv_ref, we_ref, be_ref,
                     wpe_ref, bpe_ref, wple_ref, bple_ref, sc_ref,
                     wqkv_ref, bqkv_ref, alpha_ref, ctx_ref, x0_ref):
    x0 = _embed_rows(src_ref[...], frac_ref[...], cbfv_ref, we_ref, be_ref,
                     wpe_ref, bpe_ref, wple_ref, bple_ref, sc_ref)
    x0_ref[...] = x0
    _attn_core(x0, frac_ref, fracr_ref, wqkv_ref, bqkv_ref, alpha_ref, ctx_ref)


def _attn_first(src2d, frac2d, fracr, p, wqkv, bqkv):
    grid = (N_TOK // ATT_ROWS,)
    half = D_MODEL // 2
    scalars = jnp.stack([p['emb_scaler'], p['pos_scaler'],
                         p['pos_scaler_log']]).reshape(1, 3)
    full = lambda *shape: pl.BlockSpec(shape, lambda i: (0,) * len(shape))
    return pl.pallas_call(
        _attn_first_body,
        grid=grid,
        in_specs=[
            pl.BlockSpec((ATT_ROWS, 1), lambda i: (i, 0)),
            pl.BlockSpec((ATT_ROWS, 1), lambda i: (i, 0)),
            pl.BlockSpec((1, 1, ATT_ROWS), lambda i: (i, 0, 0)),
            full(VOCAB, FEAT),
            full(FEAT, D_MODEL),
            full(1, D_MODEL),
            full(N_BASIS, half),
            full(1, half),
            full(N_BASIS, half),
            full(1, half),
            full(1, 3),
            full(D_MODEL, 3 * D_MODEL),
            full(1, 3 * D_MODEL),
            full(1, 1),
        ],
        out_specs=[
            pl.BlockSpec((ATT_ROWS, D_MODEL), lambda i: (i, 0)),
            pl.BlockSpec((ATT_ROWS, D_MODEL), lambda i: (i, 0)),
        ],
        out_shape=[
            jax.ShapeDtypeStruct((N_TOK, D_MODEL), jnp.float32),
            jax.ShapeDtypeStruct((N_TOK, D_MODEL), jnp.float32),
        ],
    )(src2d, frac2d, fracr, p['cbfv'], p['We'], p['be'].reshape(1, -1),
      p['W_pe'], p['b_pe'].reshape(1, -1), p['W_ple'],
      p['b_ple'].reshape(1, -1), scalars, wqkv, bqkv,
      p['alpha'].reshape(1, 1))


def _attn_mid_body(o1_ref, o2_ref, w1_ref, w2_ref, xres_ref, g2_ref, b2_ref,
                   frac_ref, fracr_ref, wqkv_ref, bqkv_ref, alpha_ref,
                   ctx_ref, x_ref):
    y = (w1_ref[...] * o1_ref[...] + w2_ref[...] * o2_ref[...]
         + xres_ref[...])
    m = jnp.mean(y, axis=1, keepdims=True)
    v = jnp.mean((y - m) ** 2, axis=1, keepdims=True)
    x = (y - m) / jnp.sqrt(v + 1e-5) * g2_ref[...] + b2_ref[...]
    x_ref[...] = x
    _attn_core(x, frac_ref, fracr_ref, wqkv_ref, bqkv_ref, alpha_ref, ctx_ref)


def _attn_mid(o1, o2, w1, w2, xres, frac2d, fracr, p, wqkv, bqkv):
    grid = (N_TOK // ATT_ROWS,)
    full = lambda *shape: pl.BlockSpec(shape, lambda i: (0,) * len(shape))
    return pl.pallas_call(
        _attn_mid_body,
        grid=grid,
        in_specs=[
            pl.BlockSpec((ATT_ROWS, D_MODEL), lambda i: (i, 0)),
            pl.BlockSpec((ATT_ROWS, D_MODEL), lambda i: (i, 0)),
            pl.BlockSpec((ATT_ROWS, 1), lambda i: (i, 0)),
            pl.BlockSpec((ATT_ROWS, 1), lambda i: (i, 0)),
            pl.BlockSpec((ATT_ROWS, D_MODEL), lambda i: (i, 0)),
            full(1, D_MODEL),
            full(1, D_MODEL),
            pl.BlockSpec((ATT_ROWS, 1), lambda i: (i, 0)),
            pl.BlockSpec((1, 1, ATT_ROWS), lambda i: (i, 0, 0)),
            full(D_MODEL, 3 * D_MODEL),
            full(1, 3 * D_MODEL),
            full(1, 1),
        ],
        out_specs=[
            pl.BlockSpec((ATT_ROWS, D_MODEL), lambda i: (i, 0)),
            pl.BlockSpec((ATT_ROWS, D_MODEL), lambda i: (i, 0)),
        ],
        out_shape=[
            jax.ShapeDtypeStruct((N_TOK, D_MODEL), jnp.float32),
            jax.ShapeDtypeStruct((N_TOK, D_MODEL), jnp.float32),
        ],
    )(o1, o2, w1, w2, xres, p['g2'].reshape(1, -1), p['b2'].reshape(1, -1),
      frac2d, fracr, wqkv, bqkv, p['alpha'].reshape(1, 1))


def _post_body(ctx_ref, xin_ref, wo_ref, bo_ref, g1_ref, b1_ref, wg_ref,
               bg_ref, x1_ref, xp_ref, i1_ref, i2_ref, w1_ref, w2_ref):
    y = jnp.dot(ctx_ref[...].astype(jnp.bfloat16), wo_ref[...],
                preferred_element_type=jnp.float32)
    y = y + bo_ref[...] + xin_ref[...]
    m = jnp.mean(y, axis=1, keepdims=True)
    v = jnp.mean((y - m) ** 2, axis=1, keepdims=True)
    x1 = (y - m) / jnp.sqrt(v + 1e-5) * g1_ref[...] + b1_ref[...]
    x1_ref[...] = x1
    half = D_MODEL // 2
    hi = jax.lax.bitcast_convert_type(
        x1[:, :half].astype(jnp.bfloat16).astype(jnp.float32), jnp.uint32)
    lo = jax.lax.bitcast_convert_type(
        x1[:, half:].astype(jnp.bfloat16).astype(jnp.float32), jnp.uint32)
    xp_ref[...] = jax.lax.bitcast_convert_type(hi | (lo >> 16), jnp.int32)
    logits = jnp.dot(x1.astype(jnp.bfloat16), wg_ref[...],
                     preferred_element_type=jnp.float32)
    logits = logits + bg_ref[...]
    lm = jnp.max(logits, axis=1, keepdims=True)
    le = jnp.exp(logits - lm)
    probs = le / jnp.sum(le, axis=1, keepdims=True)
    rows = probs.shape[0]
    lane = jax.lax.broadcasted_iota(jnp.int32, (rows, N_EXPERTS), 1)
    w1 = jnp.max(probs, axis=1, keepdims=True)
    i1 = jnp.min(jnp.where(probs == w1, lane, N_EXPERTS), axis=1, keepdims=True)
    probs2 = jnp.where(lane == i1, -1.0, probs)
    w2 = jnp.max(probs2, axis=1, keepdims=True)
    i2 = jnp.min(jnp.where(probs2 == w2, lane, N_EXPERTS), axis=1, keepdims=True)
    i1_ref[...] = i1
    i2_ref[...] = i2
    w1_ref[...] = w1
    w2_ref[...] = w2


def _post(ctx, xin, p, wo, bo, wg):
    blk = 512
    grid = (N_TOK // blk,)
    return pl.pallas_call(
        _post_body,
        grid=grid,
        in_specs=[
            pl.BlockSpec((blk, D_MODEL), lambda i: (i, 0)),
            pl.BlockSpec((blk, D_MODEL), lambda i: (i, 0)),
            pl.BlockSpec((D_MODEL, D_MODEL), lambda i: (0, 0)),
            pl.BlockSpec((1, D_MODEL), lambda i: (0, 0)),
            pl.BlockSpec((1, D_MODEL), lambda i: (0, 0)),
            pl.BlockSpec((1, D_MODEL), lambda i: (0, 0)),
            pl.BlockSpec((D_MODEL, N_EXPERTS), lambda i: (0, 0)),
            pl.BlockSpec((1, N_EXPERTS), lambda i: (0, 0)),
        ],
        out_specs=[
            pl.BlockSpec((blk, D_MODEL), lambda i: (i, 0)),
            pl.BlockSpec((blk, D_MODEL // 2), lambda i: (i, 0)),
            pl.BlockSpec((blk, 1), lambda i: (i, 0)),
            pl.BlockSpec((blk, 1), lambda i: (i, 0)),
            pl.BlockSpec((blk, 1), lambda i: (i, 0)),
            pl.BlockSpec((blk, 1), lambda i: (i, 0)),
        ],
        out_shape=[
            jax.ShapeDtypeStruct((N_TOK, D_MODEL), jnp.float32),
            jax.ShapeDtypeStruct((N_TOK, D_MODEL // 2), jnp.int32),
            jax.ShapeDtypeStruct((N_TOK, 1), jnp.int32),
            jax.ShapeDtypeStruct((N_TOK, 1), jnp.int32),
            jax.ShapeDtypeStruct((N_TOK, 1), jnp.float32),
            jax.ShapeDtypeStruct((N_TOK, 1), jnp.float32),
        ],
    )(ctx, xin, wo, bo, p['g1'].reshape(1, -1), p['b1'].reshape(1, -1),
      wg, p['bg'].reshape(1, -1))


def _cumsum_rows(a):
    n, w = a.shape
    sh = 1
    while sh < n:
        a = a + jnp.concatenate(
            [jnp.zeros((sh, w), a.dtype), a[:-sh]], axis=0)
        sh *= 2
    return a


def _pos_body(i1_ref, i2_ref, p1_ref, p2_ref, meta_ref):
    lane = jax.lax.broadcasted_iota(jnp.int32, (N_TOK, N_EXPERTS), 1)
    h1 = (i1_ref[...] == lane).astype(jnp.int32)
    h2 = (i2_ref[...] == lane).astype(jnp.int32)
    c1 = _cumsum_rows(h1)
    c2 = _cumsum_rows(h2)
    cnt1 = c1[N_TOK - 1:N_TOK, :]
    cnt2 = c2[N_TOK - 1:N_TOK, :]
    counts = cnt1 + cnt2
    lane8 = jax.lax.broadcasted_iota(jnp.int32, (1, N_EXPERTS), 1)

    offs_row = jnp.zeros((1, N_EXPERTS), jnp.int32)
    off = jnp.int32(0)
    off_e = []
    cnt_e = []
    cnt1_e = []
    for e in range(N_EXPERTS):
        ce = jnp.sum(jnp.where(lane8 == e, counts, 0))
        c1e = jnp.sum(jnp.where(lane8 == e, cnt1, 0))
        off_e.append(off)
        cnt_e.append(ce)
        cnt1_e.append(c1e)
        offs_row = offs_row + jnp.where(lane8 == e, off, 0)
        off = off + ce

    cnt1_row = cnt1
    p1_ref[...] = jnp.sum(h1 * (offs_row + c1 - h1), axis=1, keepdims=True)
    p2_ref[...] = jnp.sum(h2 * (offs_row + cnt1_row + c2 - h2), axis=1,
                          keepdims=True)

    lane_s = jax.lax.broadcasted_iota(jnp.int32, (1, NSTEP), 1)
    rb_row = jnp.zeros((1, NSTEP), jnp.int32)
    e_row = jnp.zeros((1, NSTEP), jnp.int32)
    st_row = jnp.zeros((1, NSTEP), jnp.int32)
    en_row = jnp.zeros((1, NSTEP), jnp.int32)
    any_row = jnp.zeros((1, NSTEP), jnp.int32)
    cum = jnp.int32(0)
    for e in range(N_EXPERTS):
        start = off_e[e]
        end = off_e[e] + cnt_e[e]
        nonempty = cnt_e[e] > 0
        fb = start // GBLK
        lb = jnp.where(nonempty, (end - 1) // GBLK, 0)
        nb = jnp.where(nonempty, lb - fb + 1, 0)
        active = (lane_s >= cum) & (lane_s < cum + nb)
        rb_here = fb + (lane_s - cum)
        rb_row = rb_row + jnp.where(active, rb_here, 0)
        e_row = e_row + jnp.where(active, e, 0)
        st_row = st_row + jnp.where(active, jnp.maximum(start, rb_here * GBLK), 0)
        en_row = en_row + jnp.where(active, jnp.minimum(end, (rb_here + 1) * GBLK), 0)
        any_row = any_row + active.astype(jnp.int32)
        cum = cum + nb
    rb_row = jnp.where(any_row > 0, rb_row, NGB - 1)
    prev = jnp.concatenate(
        [jnp.full((1, 1), -1, jnp.int32), rb_row[:, :NSTEP - 1]], axis=1)
    first_row = (rb_row != prev).astype(jnp.int32)
    meta_ref[...] = jnp.concatenate(
        [rb_row, e_row, first_row, st_row, en_row], axis=0)


def _positions(i1, i2):
    return pl.pallas_call(
        _pos_body,
        grid=(1,),
        in_specs=[
            pl.BlockSpec((N_TOK, 1), lambda i: (0, 0)),
            pl.BlockSpec((N_TOK, 1), lambda i: (0, 0)),
        ],
        out_specs=[
            pl.BlockSpec((N_TOK, 1), lambda i: (0, 0)),
            pl.BlockSpec((N_TOK, 1), lambda i: (0, 0)),
            pl.BlockSpec((5, NSTEP), lambda i: (0, 0)),
        ],
        out_shape=[
            jax.ShapeDtypeStruct((N_TOK, 1), jnp.int32),
            jax.ShapeDtypeStruct((N_TOK, 1), jnp.int32),
            jax.ShapeDtypeStruct((5, NSTEP), jnp.int32),
        ],
    )(i1, i2)


_SC_CH = 64


def _sc_mesh():
    return plsc.VectorSubcoreMesh(core_axis_name="c", subcore_axis_name="s",
                                  num_cores=_NC, num_subcores=_NS)


def _sc_dispatch(xp, p1f, p2f):
    tok_per_w = N_TOK // _NW
    width = D_MODEL // 2

    def body(x_hbm, p1_hbm, p2_hbm, xs_hbm, idx_v, rows_v, sem):
        wid = jax.lax.axis_index("s") * _NC + jax.lax.axis_index("c")
        base0 = wid * tok_per_w
        for c in range(tok_per_w // _SC_CH):
            base = base0 + c * _SC_CH
            pltpu.sync_copy(x_hbm.at[pl.ds(base, _SC_CH)], rows_v)
            pltpu.sync_copy(p1_hbm.at[pl.ds(base, _SC_CH)], idx_v)
            pltpu.async_copy(rows_v, xs_hbm.at[idx_v], sem).wait()
            pltpu.sync_copy(p2_hbm.at[pl.ds(base, _SC_CH)], idx_v)
            pltpu.async_copy(rows_v, xs_hbm.at[idx_v], sem).wait()

    f = pl.kernel(
        body,
        out_type=jax.ShapeDtypeStruct((N_SLOT, width), jnp.int32),
        mesh=_sc_mesh(),
        scratch_types=[
            pltpu.VMEM((_SC_CH,), jnp.int32),
            pltpu.VMEM((_SC_CH, width), jnp.int32),
            pltpu.SemaphoreType.DMA,
        ],
    )
    return f(xp, p1f, p2f)


def _sc_gather2(ys, p1f, p2f):
    tok_per_w = N_TOK // _NW

    def body(ys_hbm, p1_hbm, p2_hbm, o1_hbm, o2_hbm, idx_v, rows_v, sem):
        wid = jax.lax.axis_index("s") * _NC + jax.lax.axis_index("c")
        base0 = wid * tok_per_w
        for c in range(tok_per_w // _SC_CH):
            base = base0 + c * _SC_CH
            pltpu.sync_copy(p1_hbm.at[pl.ds(base, _SC_CH)], idx_v)
            pltpu.async_copy(ys_hbm.at[idx_v], rows_v, sem).wait()
            pltpu.sync_copy(rows_v, o1_hbm.at[pl.ds(base, _SC_CH)])
            pltpu.sync_copy(p2_hbm.at[pl.ds(base, _SC_CH)], idx_v)
            pltpu.async_copy(ys_hbm.at[idx_v], rows_v, sem).wait()
            pltpu.sync_copy(rows_v, o2_hbm.at[pl.ds(base, _SC_CH)])

    f = pl.kernel(
        body,
        out_type=[
            jax.ShapeDtypeStruct((N_TOK, D_MODEL), jnp.float32),
            jax.ShapeDtypeStruct((N_TOK, D_MODEL), jnp.float32),
        ],
        mesh=_sc_mesh(),
        scratch_types=[
            pltpu.VMEM((_SC_CH,), jnp.int32),
            pltpu.VMEM((_SC_CH, D_MODEL), jnp.float32),
            pltpu.SemaphoreType.DMA,
        ],
    )
    return f(ys, p1f, p2f)


def _gmm_body(meta_ref, xs_ref, we1_ref, be1_ref, we2_ref, be2_ref, ys_ref):
    s = pl.program_id(0)
    rb = meta_ref[0, s]
    first = meta_ref[2, s]
    start = meta_ref[3, s]
    end = meta_ref[4, s]
    packed = jax.lax.bitcast_convert_type(xs_ref[...], jnp.uint32)
    xhi = jax.lax.bitcast_convert_type(
        packed & jnp.uint32(0xFFFF0000), jnp.float32)
    xlo = jax.lax.bitcast_convert_type(packed << 16, jnp.float32)
    x = jnp.concatenate([xhi, xlo], axis=1).astype(jnp.bfloat16)
    h = jnp.dot(x, we1_ref[0], preferred_element_type=jnp.float32)
    h = jnp.maximum(h + be1_ref[0], 0.0)
    y = jnp.dot(h.astype(jnp.bfloat16), we2_ref[0],
                preferred_element_type=jnp.float32)
    y = y + be2_ref[0]
    gi = rb * GBLK + jax.lax.broadcasted_iota(jnp.int32, (GBLK, 1), 0)
    rowmask = (gi >= start) & (gi < end)
    contrib = jnp.where(rowmask, y, 0.0)

    @pl.when(first == 1)
    def _init():
        ys_ref[...] = contrib

    @pl.when(first == 0)
    def _acc():
        ys_ref[...] = ys_ref[...] + contrib


def _gmm(meta, xs, we1b, we2b, p):
    grid_spec = pltpu.PrefetchScalarGridSpec(
        num_scalar_prefetch=1,
        grid=(NSTEP,),
        in_specs=[
            pl.BlockSpec((GBLK, D_MODEL // 2), lambda s, m: (m[0, s], 0)),
            pl.BlockSpec((1, D_MODEL, D_FF), lambda s, m: (m[1, s], 0, 0)),
            pl.BlockSpec((1, 1, D_FF), lambda s, m: (m[1, s], 0, 0)),
            pl.BlockSpec((1, D_FF, D_MODEL), lambda s, m: (m[1, s], 0, 0)),
            pl.BlockSpec((1, 1, D_MODEL), lambda s, m: (m[1, s], 0, 0)),
        ],
        out_specs=pl.BlockSpec((GBLK, D_MODEL), lambda s, m: (m[0, s], 0)),
    )
    return pl.pallas_call(
        _gmm_body,
        grid_spec=grid_spec,
        out_shape=jax.ShapeDtypeStruct((N_SLOT, D_MODEL), jnp.float32),
    )(meta, xs, we1b, p['be1'].reshape(N_EXPERTS, 1, D_FF),
      we2b, p['be2'].reshape(N_EXPERTS, 1, D_MODEL))


def _cln2_body(o1_ref, o2_ref, w1_ref, w2_ref, xres_ref, g_ref, b_ref,
               mask_ref, out_ref):
    y = w1_ref[...] * o1_ref[...] + w2_ref[...] * o2_ref[...] + xres_ref[...]
    m = jnp.mean(y, axis=1, keepdims=True)
    v = jnp.mean((y - m) ** 2, axis=1, keepdims=True)
    x2 = (y - m) / jnp.sqrt(v + 1e-5) * g_ref[...] + b_ref[...]
    out_ref[...] = x2 * mask_ref[...]


def _combine_ln2(o1, o2, w1, w2, xres, p, mask):
    blk = 512
    return pl.pallas_call(
        _cln2_body,
        grid=(N_TOK // blk,),
        in_specs=[
            pl.BlockSpec((blk, D_MODEL), lambda i: (i, 0)),
            pl.BlockSpec((blk, D_MODEL), lambda i: (i, 0)),
            pl.BlockSpec((blk, 1), lambda i: (i, 0)),
            pl.BlockSpec((blk, 1), lambda i: (i, 0)),
            pl.BlockSpec((blk, D_MODEL), lambda i: (i, 0)),
            pl.BlockSpec((1, D_MODEL), lambda i: (0, 0)),
            pl.BlockSpec((1, D_MODEL), lambda i: (0, 0)),
            pl.BlockSpec((blk, 1), lambda i: (i, 0)),
        ],
        out_specs=pl.BlockSpec((blk, D_MODEL), lambda i: (i, 0)),
        out_shape=jax.ShapeDtypeStruct((N_TOK, D_MODEL), jnp.float32),
    )(o1, o2, w1, w2, xres, p['g2'].reshape(1, -1), p['b2'].reshape(1, -1),
      mask)


def kernel(frac, params, src):
    p = params
    frac2d = frac.reshape(N_TOK, 1)
    fracr = frac.reshape(N_TOK // ATT_ROWS, 1, ATT_ROWS)
    src2d = src.reshape(N_TOK, 1).astype(jnp.int32)
    wqkv = jnp.concatenate([p['Wq'], p['Wk'], p['Wv']], axis=1)
    bqkv = jnp.concatenate([p['bq'], p['bk'], p['bv']]).reshape(1, -1)
    ones = jnp.ones((N_TOK, 1), jnp.float32)
    finalmask = (frac2d != 0.0).astype(jnp.float32)
    we1b = p['We1'].astype(jnp.bfloat16)
    we2b = p['We2'].astype(jnp.bfloat16)
    wqkv = wqkv.astype(jnp.bfloat16)
    wob = p['Wo'].astype(jnp.bfloat16)
    wgb = p['Wg'].astype(jnp.bfloat16)

    ctx, x = _attn_first(src2d, frac2d, fracr, p, wqkv, bqkv)
    for layer in range(3):
        x1, xp, i1, i2, w1, w2 = _post(ctx, x, p, wob,
                                       p['bo'].reshape(1, -1), wgb)
        p1, p2, meta = _positions(i1, i2)
        p1f = p1.reshape(N_TOK)
        p2f = p2.reshape(N_TOK)
        xs = _sc_dispatch(xp, p1f, p2f)
        ys = _gmm(meta, xs, we1b, we2b, p)
        o1, o2 = _sc_gather2(ys, p1f, p2f)
        if layer < 2:
            ctx, x = _attn_mid(o1, o2, w1, w2, x1, frac2d, fracr, p,
                               wqkv, bqkv)
        else:
            x = _combine_ln2(o1, o2, w1, w2, x1, p, finalmask)
    return x.reshape(B, T, D_MODEL)


# speedup vs baseline: 1.8758x; 1.0013x over previous
import functools

import jax
import jax.numpy as jnp
import numpy as np
from jax.experimental import pallas as pl
from jax.experimental.pallas import tpu as pltpu
from jax.experimental.pallas import tpu_sc as plsc

D_MODEL = 1024
N_HEADS = 16
HEAD_DIM = 64
N_EXPERTS = 8
D_FF = 2048
N_BASIS = 10
DEGREE = 3
VOCAB = 120
FEAT = 200
B = 512
T = 8
N_TOK = B * T
N_SLOT = 2 * N_TOK
GBLK = 256
NGB = N_SLOT // GBLK
NSTEP = NGB + N_EXPERTS

_NC, _NS = 2, 16
_NW = _NC * _NS

_base = np.linspace(0.0, 1.0, N_BASIS + DEGREE + 1 - 2 * DEGREE)
_KNOTS = np.concatenate(
    [np.repeat(_base[:1], DEGREE), _base, np.repeat(_base[-1:], DEGREE)]
).astype(np.float64)


def _bspline_basis(f):
    nk = _KNOTS.shape[0]
    Bp = [
        jnp.where((f >= float(_KNOTS[i])) & (f < float(_KNOTS[i + 1])), 1.0, 0.0)
        for i in range(nk - 1)
    ]
    for d in range(1, DEGREE + 1):
        Bc = []
        for i in range(nk - d - 1):
            den1 = float(_KNOTS[i + d] - _KNOTS[i])
            den2 = float(_KNOTS[i + d + 1] - _KNOTS[i + 1])
            t = jnp.zeros_like(f)
            if den1 != 0.0:
                t = t + (f - float(_KNOTS[i])) / den1 * Bp[i]
            if den2 != 0.0:
                t = t + (float(_KNOTS[i + d + 1]) - f) / den2 * Bp[i + 1]
            Bc.append(t)
        Bp = Bc
    return jnp.concatenate(Bp, axis=1)


ATT_ROWS = 256


def _embed_rows(src, frac, cbfv_ref, we_ref, be_ref, wpe_ref, bpe_ref,
                wple_ref, bple_ref, sc_ref):
    rows = src.shape[0]
    oh = (src == jax.lax.broadcasted_iota(jnp.int32, (rows, VOCAB), 1)).astype(
        jnp.float32)
    feats = jnp.dot(oh, cbfv_ref[...], preferred_element_type=jnp.float32)
    x = jnp.dot(feats, we_ref[...], preferred_element_type=jnp.float32)
    x = x + be_ref[...]
    emb_scaler = sc_ref[0, 0]
    pos_scaler = sc_ref[0, 1]
    pos_scaler_log = sc_ref[0, 2]
    x = x * jnp.exp2(emb_scaler)
    pe_scaler = jnp.exp2((1.0 - pos_scaler) ** 2)
    ple_scaler = jnp.exp2((1.0 - pos_scaler_log) ** 2)

    f = jnp.clip(frac, 1e-9, 1.0)
    basis = _bspline_basis(f)
    pe = (jnp.dot(basis, wpe_ref[...], preferred_element_type=jnp.float32)
          + bpe_ref[...]) * pe_scaler
    f2 = jnp.clip(0.0025 * jnp.log2(f) ** 2, 0.0, 1.0)
    basis2 = _bspline_basis(f2)
    ple = (jnp.dot(basis2, wple_ref[...], preferred_element_type=jnp.float32)
           + bple_ref[...]) * ple_scaler
    return x + jnp.concatenate([pe, ple], axis=1)


def _attn_core(x, frac_ref, fracr_ref, wqkv_ref, bqkv_ref, alpha_ref, out_ref):
    qkv = jnp.dot(x.astype(jnp.bfloat16), wqkv_ref[...],
                  preferred_element_type=jnp.float32)
    qkv = qkv + bqkv_ref[...]
    qkvb = qkv.astype(jnp.bfloat16)
    fcol = frac_ref[...]
    frow = fracr_ref[0]
    alpha = alpha_ref[0, 0]
    R = ATT_ROWS
    diff = fcol - frow
    bias = alpha * (jnp.log1p(jnp.abs(diff)) * jnp.sign(diff))
    ii = jax.lax.broadcasted_iota(jnp.int32, (R, R), 0)
    jj = jax.lax.broadcasted_iota(jnp.int32, (R, R), 1)
    same = (ii // T) == (jj // T)
    keyok = frow != 0.0
    valid = same & keyok
    scale = HEAD_DIM ** -0.5
    for h in range(N_HEADS):
        q = qkvb[:, h * HEAD_DIM:(h + 1) * HEAD_DIM]
        k = qkvb[:, D_MODEL + h * HEAD_DIM:D_MODEL + (h + 1) * HEAD_DIM]
        v = qkvb[:, 2 * D_MODEL + h * HEAD_DIM:2 * D_MODEL + (h + 1) * HEAD_DIM]
        s = jax.lax.dot_general(q, k, (((1,), (1,)), ((), ())),
                                preferred_element_type=jnp.float32) * scale
        s = jnp.where(valid, s + bias, -1e30)
        m = jnp.max(s, axis=1, keepdims=True)
        e = jnp.exp(s - m)
        pr = e * (1.0 / jnp.sum(e, axis=1, keepdims=True))
        ctx = jnp.dot(pr.astype(jnp.bfloat16), v,
                      preferred_element_type=jnp.float32)
        out_ref[:, h * HEAD_DIM:(h + 1) * HEAD_DIM] = ctx


def _attn_first_body(src_ref, frac_ref, fracr_ref, cbfv_ref, we_ref, be_ref,
                     wpe_ref, bpe_ref, wple_ref, bple_ref, sc_ref,
                     wqkv_ref, bqkv_ref, alpha_ref, ctx_ref, x0_ref):
    x0 = _embed_rows(src_ref[...], frac_ref[...], cbfv_ref, we_ref, be_ref,
                     wpe_ref, bpe_ref, wple_ref, bple_ref, sc_ref)
    x0_ref[...] = x0
    _attn_core(x0, frac_ref, fracr_ref, wqkv_ref, bqkv_ref, alpha_ref, ctx_ref)


def _attn_first(src2d, frac2d, fracr, p, wqkv, bqkv):
    grid = (N_TOK // ATT_ROWS,)
    half = D_MODEL // 2
    scalars = jnp.stack([p['emb_scaler'], p['pos_scaler'],
                         p['pos_scaler_log']]).reshape(1, 3)
    full = lambda *shape: pl.BlockSpec(shape, lambda i: (0,) * len(shape))
    return pl.pallas_call(
        _attn_first_body,
        grid=grid,
        in_specs=[
            pl.BlockSpec((ATT_ROWS, 1), lambda i: (i, 0)),
            pl.BlockSpec((ATT_ROWS, 1), lambda i: (i, 0)),
            pl.BlockSpec((1, 1, ATT_ROWS), lambda i: (i, 0, 0)),
            full(VOCAB, FEAT),
            full(FEAT, D_MODEL),
            full(1, D_MODEL),
            full(N_BASIS, half),
            full(1, half),
            full(N_BASIS, half),
            full(1, half),
            full(1, 3),
            full(D_MODEL, 3 * D_MODEL),
            full(1, 3 * D_MODEL),
            full(1, 1),
        ],
        out_specs=[
            pl.BlockSpec((ATT_ROWS, D_MODEL), lambda i: (i, 0)),
            pl.BlockSpec((ATT_ROWS, D_MODEL), lambda i: (i, 0)),
        ],
        out_shape=[
            jax.ShapeDtypeStruct((N_TOK, D_MODEL), jnp.float32),
            jax.ShapeDtypeStruct((N_TOK, D_MODEL), jnp.float32),
        ],
    )(src2d, frac2d, fracr, p['cbfv'], p['We'], p['be'].reshape(1, -1),
      p['W_pe'], p['b_pe'].reshape(1, -1), p['W_ple'],
      p['b_ple'].reshape(1, -1), scalars, wqkv, bqkv,
      p['alpha'].reshape(1, 1))


def _attn_mid_body(o1_ref, o2_ref, w1_ref, w2_ref, xres_ref, g2_ref, b2_ref,
                   frac_ref, fracr_ref, wqkv_ref, bqkv_ref, alpha_ref,
                   ctx_ref, x_ref):
    y = (w1_ref[...] * o1_ref[...] + w2_ref[...] * o2_ref[...]
         + xres_ref[...])
    m = jnp.mean(y, axis=1, keepdims=True)
    v = jnp.mean((y - m) ** 2, axis=1, keepdims=True)
    x = (y - m) / jnp.sqrt(v + 1e-5) * g2_ref[...] + b2_ref[...]
    x_ref[...] = x
    _attn_core(x, frac_ref, fracr_ref, wqkv_ref, bqkv_ref, alpha_ref, ctx_ref)


def _attn_mid(o1, o2, w1, w2, xres, frac2d, fracr, p, wqkv, bqkv):
    grid = (N_TOK // ATT_ROWS,)
    full = lambda *shape: pl.BlockSpec(shape, lambda i: (0,) * len(shape))
    return pl.pallas_call(
        _attn_mid_body,
        grid=grid,
        in_specs=[
            pl.BlockSpec((ATT_ROWS, D_MODEL), lambda i: (i, 0)),
            pl.BlockSpec((ATT_ROWS, D_MODEL), lambda i: (i, 0)),
            pl.BlockSpec((ATT_ROWS, 1), lambda i: (i, 0)),
            pl.BlockSpec((ATT_ROWS, 1), lambda i: (i, 0)),
            pl.BlockSpec((ATT_ROWS, D_MODEL), lambda i: (i, 0)),
            full(1, D_MODEL),
            full(1, D_MODEL),
            pl.BlockSpec((ATT_ROWS, 1), lambda i: (i, 0)),
            pl.BlockSpec((1, 1, ATT_ROWS), lambda i: (i, 0, 0)),
            full(D_MODEL, 3 * D_MODEL),
            full(1, 3 * D_MODEL),
            full(1, 1),
        ],
        out_specs=[
            pl.BlockSpec((ATT_ROWS, D_MODEL), lambda i: (i, 0)),
            pl.BlockSpec((ATT_ROWS, D_MODEL), lambda i: (i, 0)),
        ],
        out_shape=[
            jax.ShapeDtypeStruct((N_TOK, D_MODEL), jnp.float32),
            jax.ShapeDtypeStruct((N_TOK, D_MODEL), jnp.float32),
        ],
    )(o1, o2, w1, w2, xres, p['g2'].reshape(1, -1), p['b2'].reshape(1, -1),
      frac2d, fracr, wqkv, bqkv, p['alpha'].reshape(1, 1))


def _post_body(ctx_ref, xin_ref, wo_ref, bo_ref, g1_ref, b1_ref, wg_ref,
               bg_ref, x1_ref, xp_ref, i1_ref, i2_ref, w1_ref, w2_ref):
    y = jnp.dot(ctx_ref[...].astype(jnp.bfloat16), wo_ref[...],
                preferred_element_type=jnp.float32)
    y = y + bo_ref[...] + xin_ref[...]
    m = jnp.mean(y, axis=1, keepdims=True)
    v = jnp.mean((y - m) ** 2, axis=1, keepdims=True)
    x1 = (y - m) / jnp.sqrt(v + 1e-5) * g1_ref[...] + b1_ref[...]
    x1_ref[...] = x1
    half = D_MODEL // 2
    hi = jax.lax.bitcast_convert_type(
        x1[:, :half].astype(jnp.bfloat16).astype(jnp.float32), jnp.uint32)
    lo = jax.lax.bitcast_convert_type(
        x1[:, half:].astype(jnp.bfloat16).astype(jnp.float32), jnp.uint32)
    xp_ref[...] = jax.lax.bitcast_convert_type(hi | (lo >> 16), jnp.int32)
    logits = jnp.dot(x1.astype(jnp.bfloat16), wg_ref[...],
                     preferred_element_type=jnp.float32)
    logits = logits + bg_ref[...]
    lm = jnp.max(logits, axis=1, keepdims=True)
    le = jnp.exp(logits - lm)
    probs = le / jnp.sum(le, axis=1, keepdims=True)
    rows = probs.shape[0]
    lane = jax.lax.broadcasted_iota(jnp.int32, (rows, N_EXPERTS), 1)
    w1 = jnp.max(probs, axis=1, keepdims=True)
    i1 = jnp.min(jnp.where(probs == w1, lane, N_EXPERTS), axis=1, keepdims=True)
    probs2 = jnp.where(lane == i1, -1.0, probs)
    w2 = jnp.max(probs2, axis=1, keepdims=True)
    i2 = jnp.min(jnp.where(probs2 == w2, lane, N_EXPERTS), axis=1, keepdims=True)
    i1_ref[...] = i1
    i2_ref[...] = i2
    w1_ref[...] = w1
    w2_ref[...] = w2


def _post(ctx, xin, p, wo, bo, wg):
    blk = 512
    grid = (N_TOK // blk,)
    return pl.pallas_call(
        _post_body,
        grid=grid,
        in_specs=[
            pl.BlockSpec((blk, D_MODEL), lambda i: (i, 0)),
            pl.BlockSpec((blk, D_MODEL), lambda i: (i, 0)),
            pl.BlockSpec((D_MODEL, D_MODEL), lambda i: (0, 0)),
            pl.BlockSpec((1, D_MODEL), lambda i: (0, 0)),
            pl.BlockSpec((1, D_MODEL), lambda i: (0, 0)),
            pl.BlockSpec((1, D_MODEL), lambda i: (0, 0)),
            pl.BlockSpec((D_MODEL, N_EXPERTS), lambda i: (0, 0)),
            pl.BlockSpec((1, N_EXPERTS), lambda i: (0, 0)),
        ],
        out_specs=[
            pl.BlockSpec((blk, D_MODEL), lambda i: (i, 0)),
            pl.BlockSpec((blk, D_MODEL // 2), lambda i: (i, 0)),
            pl.BlockSpec((blk, 1), lambda i: (i, 0)),
            pl.BlockSpec((blk, 1), lambda i: (i, 0)),
            pl.BlockSpec((blk, 1), lambda i: (i, 0)),
            pl.BlockSpec((blk, 1), lambda i: (i, 0)),
        ],
        out_shape=[
            jax.ShapeDtypeStruct((N_TOK, D_MODEL), jnp.float32),
            jax.ShapeDtypeStruct((N_TOK, D_MODEL // 2), jnp.int32),
            jax.ShapeDtypeStruct((N_TOK, 1), jnp.int32),
            jax.ShapeDtypeStruct((N_TOK, 1), jnp.int32),
            jax.ShapeDtypeStruct((N_TOK, 1), jnp.float32),
            jax.ShapeDtypeStruct((N_TOK, 1), jnp.float32),
        ],
    )(ctx, xin, wo, bo, p['g1'].reshape(1, -1), p['b1'].reshape(1, -1),
      wg, p['bg'].reshape(1, -1))


def _cumsum_rows(a):
    n, w = a.shape
    sh = 1
    while sh < n:
        a = a + jnp.concatenate(
            [jnp.zeros((sh, w), a.dtype), a[:-sh]], axis=0)
        sh *= 2
    return a


def _pos_body(i1_ref, i2_ref, p1_ref, p2_ref, meta_ref):
    lane = jax.lax.broadcasted_iota(jnp.int32, (N_TOK, N_EXPERTS), 1)
    h1 = (i1_ref[...] == lane).astype(jnp.int32)
    h2 = (i2_ref[...] == lane).astype(jnp.int32)
    c1 = _cumsum_rows(h1)
    c2 = _cumsum_rows(h2)
    cnt1 = c1[N_TOK - 1:N_TOK, :]
    cnt2 = c2[N_TOK - 1:N_TOK, :]
    counts = cnt1 + cnt2
    lane8 = jax.lax.broadcasted_iota(jnp.int32, (1, N_EXPERTS), 1)

    offs_row = jnp.zeros((1, N_EXPERTS), jnp.int32)
    off = jnp.int32(0)
    off_e = []
    cnt_e = []
    cnt1_e = []
    for e in range(N_EXPERTS):
        ce = jnp.sum(jnp.where(lane8 == e, counts, 0))
        c1e = jnp.sum(jnp.where(lane8 == e, cnt1, 0))
        off_e.append(off)
        cnt_e.append(ce)
        cnt1_e.append(c1e)
        offs_row = offs_row + jnp.where(lane8 == e, off, 0)
        off = off + ce

    cnt1_row = cnt1
    p1_ref[...] = jnp.sum(h1 * (offs_row + c1 - h1), axis=1, keepdims=True)
    p2_ref[...] = jnp.sum(h2 * (offs_row + cnt1_row + c2 - h2), axis=1,
                          keepdims=True)

    lane_s = jax.lax.broadcasted_iota(jnp.int32, (1, NSTEP), 1)
    rb_row = jnp.zeros((1, NSTEP), jnp.int32)
    e_row = jnp.zeros((1, NSTEP), jnp.int32)
    st_row = jnp.zeros((1, NSTEP), jnp.int32)
    en_row = jnp.zeros((1, NSTEP), jnp.int32)
    any_row = jnp.zeros((1, NSTEP), jnp.int32)
    cum = jnp.int32(0)
    for e in range(N_EXPERTS):
        start = off_e[e]
        end = off_e[e] + cnt_e[e]
        nonempty = cnt_e[e] > 0
        fb = start // GBLK
        lb = jnp.where(nonempty, (end - 1) // GBLK, 0)
        nb = jnp.where(nonempty, lb - fb + 1, 0)
        active = (lane_s >= cum) & (lane_s < cum + nb)
        rb_here = fb + (lane_s - cum)
        rb_row = rb_row + jnp.where(active, rb_here, 0)
        e_row = e_row + jnp.where(active, e, 0)
        st_row = st_row + jnp.where(active, jnp.maximum(start, rb_here * GBLK), 0)
        en_row = en_row + jnp.where(active, jnp.minimum(end, (rb_here + 1) * GBLK), 0)
        any_row = any_row + active.astype(jnp.int32)
        cum = cum + nb
    rb_row = jnp.where(any_row > 0, rb_row, NGB - 1)
    prev = jnp.concatenate(
        [jnp.full((1, 1), -1, jnp.int32), rb_row[:, :NSTEP - 1]], axis=1)
    first_row = (rb_row != prev).astype(jnp.int32)
    meta_ref[...] = jnp.concatenate(
        [rb_row, e_row, first_row, st_row, en_row], axis=0)


def _positions(i1, i2):
    return pl.pallas_call(
        _pos_body,
        grid=(1,),
        in_specs=[
            pl.BlockSpec((N_TOK, 1), lambda i: (0, 0)),
            pl.BlockSpec((N_TOK, 1), lambda i: (0, 0)),
        ],
        out_specs=[
            pl.BlockSpec((N_TOK, 1), lambda i: (0, 0)),
            pl.BlockSpec((N_TOK, 1), lambda i: (0, 0)),
            pl.BlockSpec((5, NSTEP), lambda i: (0, 0)),
        ],
        out_shape=[
            jax.ShapeDtypeStruct((N_TOK, 1), jnp.int32),
            jax.ShapeDtypeStruct((N_TOK, 1), jnp.int32),
            jax.ShapeDtypeStruct((5, NSTEP), jnp.int32),
        ],
    )(i1, i2)


_SC_CH = 64


def _sc_mesh():
    return plsc.VectorSubcoreMesh(core_axis_name="c", subcore_axis_name="s",
                                  num_cores=_NC, num_subcores=_NS)


def _sc_dispatch(xp, p1f, p2f):
    tok_per_w = N_TOK // _NW
    width = D_MODEL // 2

    def body(x_hbm, p1_hbm, p2_hbm, xs_hbm, idx_v, rows_v, sem):
        wid = jax.lax.axis_index("s") * _NC + jax.lax.axis_index("c")
        base0 = wid * tok_per_w
        for c in range(tok_per_w // _SC_CH):
            base = base0 + c * _SC_CH
            pltpu.sync_copy(x_hbm.at[pl.ds(base, _SC_CH)], rows_v)
            pltpu.sync_copy(p1_hbm.at[pl.ds(base, _SC_CH)], idx_v)
            pltpu.async_copy(rows_v, xs_hbm.at[idx_v], sem).wait()
            pltpu.sync_copy(p2_hbm.at[pl.ds(base, _SC_CH)], idx_v)
            pltpu.async_copy(rows_v, xs_hbm.at[idx_v], sem).wait()

    f = pl.kernel(
        body,
        out_type=jax.ShapeDtypeStruct((N_SLOT, width), jnp.int32),
        mesh=_sc_mesh(),
        scratch_types=[
            pltpu.VMEM((_SC_CH,), jnp.int32),
            pltpu.VMEM((_SC_CH, width), jnp.int32),
            pltpu.SemaphoreType.DMA,
        ],
    )
    return f(xp, p1f, p2f)


def _sc_gather2(ys, p1f, p2f):
    tok_per_w = N_TOK // _NW

    def body(ys_hbm, p1_hbm, p2_hbm, o1_hbm, o2_hbm, idx_v, rows_v, sem):
        wid = jax.lax.axis_index("s") * _NC + jax.lax.axis_index("c")
        base0 = wid * tok_per_w
        for c in range(tok_per_w // _SC_CH):
            base = base0 + c * _SC_CH
            pltpu.sync_copy(p1_hbm.at[pl.ds(base, _SC_CH)], idx_v)
            pltpu.async_copy(ys_hbm.at[idx_v], rows_v, sem).wait()
            pltpu.sync_copy(rows_v, o1_hbm.at[pl.ds(base, _SC_CH)])
            pltpu.sync_copy(p2_hbm.at[pl.ds(base, _SC_CH)], idx_v)
            pltpu.async_copy(ys_hbm.at[idx_v], rows_v, sem).wait()
            pltpu.sync_copy(rows_v, o2_hbm.at[pl.ds(base, _SC_CH)])

    f = pl.kernel(
        body,
        out_type=[
            jax.ShapeDtypeStruct((N_TOK, D_MODEL), jnp.float32),
            jax.ShapeDtypeStruct((N_TOK, D_MODEL), jnp.float32),
        ],
        mesh=_sc_mesh(),
        scratch_types=[
            pltpu.VMEM((_SC_CH,), jnp.int32),
            pltpu.VMEM((_SC_CH, D_MODEL), jnp.float32),
            pltpu.SemaphoreType.DMA,
        ],
    )
    return f(ys, p1f, p2f)


def _gmm_body(meta_ref, xs_ref, we1_ref, be1_ref, we2_ref, be2_ref, ys_ref):
    s = pl.program_id(0)
    rb = meta_ref[0, s]
    first = meta_ref[2, s]
    start = meta_ref[3, s]
    end = meta_ref[4, s]
    packed = jax.lax.bitcast_convert_type(xs_ref[...], jnp.uint32)
    xhi = jax.lax.bitcast_convert_type(
        packed & jnp.uint32(0xFFFF0000), jnp.float32)
    xlo = jax.lax.bitcast_convert_type(packed << 16, jnp.float32)
    x = jnp.concatenate([xhi, xlo], axis=1).astype(jnp.bfloat16)
    h = jnp.dot(x, we1_ref[0], preferred_element_type=jnp.float32)
    h = jnp.maximum(h + be1_ref[0], 0.0)
    y = jnp.dot(h.astype(jnp.bfloat16), we2_ref[0],
                preferred_element_type=jnp.float32)
    y = y + be2_ref[0]
    gi = rb * GBLK + jax.lax.broadcasted_iota(jnp.int32, (GBLK, 1), 0)
    rowmask = (gi >= start) & (gi < end)
    contrib = jnp.where(rowmask, y, 0.0)

    @pl.when(first == 1)
    def _init():
        ys_ref[...] = contrib

    @pl.when(first == 0)
    def _acc():
        ys_ref[...] = ys_ref[...] + contrib


def _gmm(meta, xs, we1b, we2b, p):
    grid_spec = pltpu.PrefetchScalarGridSpec(
        num_scalar_prefetch=1,
        grid=(NSTEP,),
        in_specs=[
            pl.BlockSpec((GBLK, D_MODEL // 2), lambda s, m: (m[0, s], 0)),
            pl.BlockSpec((1, D_MODEL, D_FF), lambda s, m: (m[1, s], 0, 0)),
            pl.BlockSpec((1, 1, D_FF), lambda s, m: (m[1, s], 0, 0)),
            pl.BlockSpec((1, D_FF, D_MODEL), lambda s, m: (m[1, s], 0, 0)),
            pl.BlockSpec((1, 1, D_MODEL), lambda s, m: (m[1, s], 0, 0)),
        ],
        out_specs=pl.BlockSpec((GBLK, D_MODEL), lambda s, m: (m[0, s], 0)),
    )
    return pl.pallas_call(
        _gmm_body,
        grid_spec=grid_spec,
        out_shape=jax.ShapeDtypeStruct((N_SLOT, D_MODEL), jnp.float32),
    )(meta, xs, we1b, p['be1'].reshape(N_EXPERTS, 1, D_FF),
      we2b, p['be2'].reshape(N_EXPERTS, 1, D_MODEL))


def _cln2_body(o1_ref, o2_ref, w1_ref, w2_ref, xres_ref, g_ref, b_ref,
               mask_ref, out_ref):
    y = w1_ref[...] * o1_ref[...] + w2_ref[...] * o2_ref[...] + xres_ref[...]
    m = jnp.mean(y, axis=1, keepdims=True)
    v = jnp.mean((y - m) ** 2, axis=1, keepdims=True)
    x2 = (y - m) / jnp.sqrt(v + 1e-5) * g_ref[...] + b_ref[...]
    out_ref[...] = x2 * mask_ref[...]


def _combine_ln2(o1, o2, w1, w2, xres, p, mask):
    blk = 512
    return pl.pallas_call(
        _cln2_body,
        grid=(N_TOK // blk,),
        in_specs=[
            pl.BlockSpec((blk, D_MODEL), lambda i: (i, 0)),
            pl.BlockSpec((blk, D_MODEL), lambda i: (i, 0)),
            pl.BlockSpec((blk, 1), lambda i: (i, 0)),
            pl.BlockSpec((blk, 1), lambda i: (i, 0)),
            pl.BlockSpec((blk, D_MODEL), lambda i: (i, 0)),
            pl.BlockSpec((1, D_MODEL), lambda i: (0, 0)),
            pl.BlockSpec((1, D_MODEL), lambda i: (0, 0)),
            pl.BlockSpec((blk, 1), lambda i: (i, 0)),
        ],
        out_specs=pl.BlockSpec((blk, D_MODEL), lambda i: (i, 0)),
        out_shape=jax.ShapeDtypeStruct((N_TOK, D_MODEL), jnp.float32),
    )(o1, o2, w1, w2, xres, p['g2'].reshape(1, -1), p['b2'].reshape(1, -1),
      mask)


def kernel(frac, params, src):
    p = params
    frac2d = frac.reshape(N_TOK, 1)
    fracr = frac.reshape(N_TOK // ATT_ROWS, 1, ATT_ROWS)
    src2d = src.reshape(N_TOK, 1).astype(jnp.int32)
    wqkv = jnp.concatenate([p['Wq'], p['Wk'], p['Wv']], axis=1)
    bqkv = jnp.concatenate([p['bq'], p['bk'], p['bv']]).reshape(1, -1)
    finalmask = (frac2d != 0.0).astype(jnp.float32)
    we1b = p['We1'].astype(jnp.bfloat16)
    we2b = p['We2'].astype(jnp.bfloat16)
    wqkv = wqkv.astype(jnp.bfloat16)
    wob = p['Wo'].astype(jnp.bfloat16)
    wgb = p['Wg'].astype(jnp.bfloat16)

    ctx, x = _attn_first(src2d, frac2d, fracr, p, wqkv, bqkv)
    for layer in range(3):
        x1, xp, i1, i2, w1, w2 = _post(ctx, x, p, wob,
                                       p['bo'].reshape(1, -1), wgb)
        p1, p2, meta = _positions(i1, i2)
        p1f = p1.reshape(N_TOK)
        p2f = p2.reshape(N_TOK)
        xs = _sc_dispatch(xp, p1f, p2f)
        ys = _gmm(meta, xs, we1b, we2b, p)
        o1, o2 = _sc_gather2(ys, p1f, p2f)
        if layer < 2:
            ctx, x = _attn_mid(o1, o2, w1, w2, x1, frac2d, fracr, p,
                               wqkv, bqkv)
        else:
            x = _combine_ln2(o1, o2, w1, w2, x1, p, finalmask)
    return x.reshape(B, T, D_MODEL)
